# baseline (speedup 1.0000x reference)
import functools

import numpy as np

import jax
import jax.numpy as jnp
from jax import lax
from jax.experimental import pallas as pl
from jax.experimental.pallas import tpu as pltpu
from jax.experimental.pallas import tpu_sc as plsc

_B, _S, _H, _NH, _HD, _F = 2, 4096, 1024, 16, 64, 2816
_EPS = 1e-5
_THETA = 10000.0
_BQ = 256
_BK = 512
_NQ = _S // _BQ
_TRASH = _B * _S
_NTILES = 32
_RPT = _B * _S // _NTILES
_SUB = 32
_NCH = _RPT // _SUB

_INTERPRET = False


def _sc_index_build(mask_i32):
    mesh = plsc.VectorSubcoreMesh(core_axis_name="c", subcore_axis_name="s", num_cores=2, num_subcores=16)

    @functools.partial(
        pl.kernel,
        out_type=(
            jax.ShapeDtypeStruct((_B, _S), jnp.int32),
            jax.ShapeDtypeStruct((_B, 16), jnp.int32),
        ),
        mesh=mesh,
        scratch_types=[
            pltpu.VMEM((_S,), jnp.int32),
            pltpu.VMEM((_S,), jnp.int32),
            pltpu.VMEM((16,), jnp.int32),
        ],
        compiler_params=pltpu.CompilerParams(needs_layout_passes=False),
        interpret=_INTERPRET,
    )
    def k(mask_hbm, gidx_hbm, lens_hbm, mask_v, gidx_v, lens_v):
        wid = lax.axis_index("s") * 2 + lax.axis_index("c")

        @pl.when(wid == 0)
        def _():
            def batch_body(b, _):
                pltpu.sync_copy(mask_hbm.at[b], mask_v)
                neg1 = jnp.full((16,), -1, jnp.int32)

                def initb(i, c):
                    gidx_v[pl.ds(i * 16, 16)] = neg1
                    return c

                lax.fori_loop(0, _S // 16, initb, 0)
                base = b * _S

                def chunk(i, carry):
                    m = mask_v[pl.ds(i * 16, 16)]
                    mb = m != 0
                    c = plsc.cumsum(m)
                    rank = c - 1 + carry
                    tvec = lax.iota(jnp.int32, 16) + i * 16 + base
                    plsc.store_scatter(gidx_v, [rank], tvec, mask=mb)
                    return carry + jnp.sum(m)

                ln = lax.fori_loop(0, _S // 16, chunk, jnp.int32(0))
                pltpu.sync_copy(gidx_v, gidx_hbm.at[b])
                lens_v[...] = jnp.zeros((16,), jnp.int32) + ln
                pltpu.sync_copy(lens_v, lens_hbm.at[b])
                return 0

            lax.fori_loop(0, _B, batch_body, 0)

    return k(mask_i32)


def _sc_gather(hid_flat, gidx_flat):
    mesh = plsc.VectorSubcoreMesh(core_axis_name="c", subcore_axis_name="s", num_cores=2, num_subcores=16)

    @functools.partial(
        pl.kernel,
        out_type=jax.ShapeDtypeStruct((_B * _S, _H), jnp.float32),
        mesh=mesh,
        scratch_types=[
            pltpu.VMEM((_RPT,), jnp.int32),
            pltpu.VMEM((_SUB, _H), jnp.float32),
            pltpu.VMEM((_SUB, _H), jnp.float32),
            pltpu.VMEM((_SUB, _H), jnp.float32),
            pltpu.SemaphoreType.DMA,
            pltpu.SemaphoreType.DMA,
            pltpu.SemaphoreType.DMA,
            pltpu.SemaphoreType.DMA,
            pltpu.SemaphoreType.DMA,
            pltpu.SemaphoreType.DMA,
        ],
        interpret=_INTERPRET,
    )
    def k(hid_hbm, gidx_hbm, out_hbm, idx_all, buf0, buf1, buf2,
          sg0, sg1, sg2, sw0, sw1, sw2):
        wid = lax.axis_index("s") * 2 + lax.axis_index("c")
        base = wid * _RPT
        bbase = (base // _S) * _S
        pltpu.sync_copy(gidx_hbm.at[pl.ds(base, _RPT)], idx_all)
        for t in range(_RPT // 16):
            g = idx_all[pl.ds(t * 16, 16)]
            idx_all[pl.ds(t * 16, 16)] = jnp.where(g < 0, bbase, g)
        bufs = (buf0, buf1, buf2)
        sgs = (sg0, sg1, sg2)
        sws = (sw0, sw1, sw2)
        gd = [None, None, None]
        wd = [None, None, None]

        def fire_g(j):
            gd[j % 3] = pltpu.async_copy(
                hid_hbm.at[idx_all.at[pl.ds(j * _SUB, _SUB)]],
                bufs[j % 3], sgs[j % 3])

        def fire_w(j):
            wd[j % 3] = pltpu.async_copy(
                bufs[j % 3], out_hbm.at[pl.ds(base + j * _SUB, _SUB)],
                sws[j % 3])

        for j in range(3):
            fire_g(j)
        for j in range(_NCH):
            gd[j % 3].wait()
            fire_w(j)
            if j + 3 < _NCH:
                wd[j % 3].wait()
                fire_g(j + 3)
        for j in range(_NCH - 3, _NCH):
            wd[j % 3].wait()

    return k(hid_flat, gidx_flat)


def _sc_scatter(hid_flat, lo_flat, mask_flat, gidx_flat):
    mesh = plsc.VectorSubcoreMesh(core_axis_name="c", subcore_axis_name="s", num_cores=2, num_subcores=16)

    @functools.partial(
        pl.kernel,
        out_type=jax.ShapeDtypeStruct((_B * _S + 8, _H), jnp.float32),
        mesh=mesh,
        scratch_types=[
            pltpu.VMEM((_RPT,), jnp.int32),
            pltpu.VMEM((_NCH, _SUB), jnp.int32),
            pltpu.VMEM((_NCH, _SUB), jnp.int32),
            pltpu.VMEM((_SUB, _H), jnp.float32),
            pltpu.VMEM((_SUB, _H), jnp.float32),
            pltpu.VMEM((_SUB, _H), jnp.float32),
            pltpu.SemaphoreType.DMA,
            pltpu.SemaphoreType.DMA,
            pltpu.SemaphoreType.DMA,
            pltpu.SemaphoreType.DMA,
            pltpu.SemaphoreType.DMA,
            pltpu.SemaphoreType.DMA,
        ],
        interpret=_INTERPRET,
    )
    def k(hid_hbm, lo_hbm, mask_hbm, gidx_hbm, out_hbm, tmp, idxA, idxB,
          buf0, buf1, buf2, sl0, sl1, sl2, ss0, ss1, ss2):
        wid = lax.axis_index("s") * 2 + lax.axis_index("c")
        base = wid * _RPT

        pltpu.sync_copy(mask_hbm.at[pl.ds(base, _RPT)], tmp)
        for t in range(_RPT // 16):
            m = tmp[pl.ds(t * 16, 16)]
            tvec = lax.iota(jnp.int32, 16) + (base + t * 16)
            idxA[t // 2, pl.ds((t % 2) * 16, 16)] = jnp.where(
                m != 0, _TRASH, tvec)
        pltpu.sync_copy(gidx_hbm.at[pl.ds(base, _RPT)], tmp)
        for t in range(_RPT // 16):
            g = tmp[pl.ds(t * 16, 16)]
            idxB[t // 2, pl.ds((t % 2) * 16, 16)] = jnp.where(
                g < 0, _TRASH, g)

        jobs = [(hid_hbm, idxA, j) for j in range(_NCH)] + \
               [(lo_hbm, idxB, j) for j in range(_NCH)]
        nj = len(jobs)
        bufs = (buf0, buf1, buf2)
        sls = (sl0, sl1, sl2)
        sss = (ss0, ss1, ss2)
        ld = [None, None, None]
        sd = [None, None, None]

        def fire_l(i):
            src, _, j = jobs[i]
            ld[i % 3] = pltpu.async_copy(
                src.at[pl.ds(base + j * _SUB, _SUB)], bufs[i % 3], sls[i % 3])

        def fire_s(i):
            _, idxr, j = jobs[i]
            sd[i % 3] = pltpu.async_copy(
                bufs[i % 3], out_hbm.at[idxr.at[j]], sss[i % 3])

        for i in range(3):
            fire_l(i)
        for i in range(nj):
            ld[i % 3].wait()
            fire_s(i)
            if i + 3 < nj:
                sd[i % 3].wait()
                fire_l(i + 3)
        for i in range(nj - 3, nj):
            sd[i % 3].wait()

    return k(hid_flat, lo_flat, mask_flat, gidx_flat)


def _qkv_body(lens_ref, hs_ref, pos_ref, w_ref, g_ref, q_ref, k_ref, v_ref):
    b = pl.program_id(0)
    qi = pl.program_id(1)
    ln = lens_ref[b, 0]

    @pl.when(qi * _BQ < ln)
    def _():
        x = hs_ref[0]
        var = jnp.mean(x * x, axis=-1, keepdims=True)
        xn = (x * lax.rsqrt(var + _EPS)) * g_ref[0]
        qkv = jnp.dot(xn.astype(jnp.bfloat16), w_ref[...],
                      preferred_element_type=jnp.float32)
        pos = pos_ref[0].astype(jnp.float32) - b * float(_S)
        l_idx = lax.broadcasted_iota(jnp.int32, (1, _H), 1)
        jmod = (l_idx % 32).astype(jnp.float32)
        invf = jnp.exp(jmod * (-np.log(_THETA) / 32.0))
        ang = pos * invf
        c = jnp.cos(ang)
        s = jnp.sin(ang)
        sel = (l_idx % 64) < 32

        def rope(t):
            xp = jnp.concatenate([t[:, 32:], t[:, :32]], axis=1)
            xm = jnp.concatenate([t[:, -32:], t[:, :-32]], axis=1)
            return jnp.where(sel, -xp, xm)

        qp = qkv[:, :_H]
        kp = qkv[:, _H:2 * _H]
        q_ref[0] = (qp * c + rope(qp) * s).astype(jnp.bfloat16)
        k_ref[0] = (kp * c + rope(kp) * s).astype(jnp.bfloat16)
        v_ref[0] = qkv[:, 2 * _H:].astype(jnp.bfloat16)

    @pl.when(qi * _BQ >= ln)
    def _():
        z = jnp.zeros((_BQ, _H), jnp.bfloat16)
        q_ref[0] = z
        k_ref[0] = z
        v_ref[0] = z


def _qkv_call(lens_x, hs_c, pos3, wqkv, g1):
    grid_spec = pltpu.PrefetchScalarGridSpec(
        num_scalar_prefetch=1,
        grid=(_B, _NQ),
        in_specs=[
            pl.BlockSpec((1, _BQ, _H), lambda b, qi, L: (b, qi, 0)),
            pl.BlockSpec((1, _BQ, 1), lambda b, qi, L: (b * _NQ + qi, 0, 0)),
            pl.BlockSpec((_H, 3 * _H), lambda b, qi, L: (0, 0)),
            pl.BlockSpec((1, _H), lambda b, qi, L: (0, 0)),
        ],
        out_specs=[
            pl.BlockSpec((1, _BQ, _H), lambda b, qi, L: (b, qi, 0)),
            pl.BlockSpec((1, _BQ, _H), lambda b, qi, L: (b, qi, 0)),
            pl.BlockSpec((1, _BQ, _H), lambda b, qi, L: (b, qi, 0)),
        ],
    )
    shp = jax.ShapeDtypeStruct((_B, _S, _H), jnp.bfloat16)
    return pl.pallas_call(
        _qkv_body,
        grid_spec=grid_spec,
        out_shape=[shp, shp, shp],
        compiler_params=pltpu.CompilerParams(
            dimension_semantics=("parallel", "parallel")),
        interpret=_INTERPRET,
    )(lens_x, hs_c, pos3, wqkv, g1)


def _attn_body(lens_ref, q_ref, k_ref, v_ref, o_ref):
    b = pl.program_id(0)
    qi = pl.program_id(2)
    start = qi * _BQ
    ln = lens_ref[b, 0]

    @pl.when(start < ln)
    def _():
        q = q_ref[0, 0]
        scale = 1.0 / np.sqrt(_HD)

        def upd(s, m, l, acc, vblk):
            m_new = jnp.maximum(m, jnp.max(s, axis=1, keepdims=True))
            alpha = jnp.exp(m - m_new)
            p = jnp.exp(s - m_new)
            l_new = l * alpha + jnp.sum(p, axis=1, keepdims=True)
            acc_new = acc * alpha + jnp.dot(p.astype(jnp.bfloat16), vblk,
                                            preferred_element_type=jnp.float32)
            return m_new, l_new, acc_new

        def kb_body(kb, carry):
            m, l, acc = carry
            kblk = k_ref[0, 0, pl.ds(kb * _BK, _BK), :]
            vblk = v_ref[0, 0, pl.ds(kb * _BK, _BK), :]
            s = lax.dot_general(q, kblk, (((1,), (1,)), ((), ())),
                                preferred_element_type=jnp.float32) * scale
            return upd(s, m, l, acc, vblk)

        ndiag = start // _BK
        mi = jnp.full((_BQ, 1), -1e30, jnp.float32)
        li = jnp.zeros((_BQ, 1), jnp.float32)
        ai = jnp.zeros((_BQ, _HD), jnp.float32)
        m, l, acc = lax.fori_loop(0, ndiag, kb_body, (mi, li, ai))

        kblk = k_ref[0, 0, pl.ds(ndiag * _BK, _BK), :]
        vblk = v_ref[0, 0, pl.ds(ndiag * _BK, _BK), :]
        s = lax.dot_general(q, kblk, (((1,), (1,)), ((), ())),
                            preferred_element_type=jnp.float32) * scale
        row = start + lax.broadcasted_iota(jnp.int32, (_BQ, 1), 0)
        col = ndiag * _BK + lax.broadcasted_iota(jnp.int32, (1, _BK), 1)
        s = jnp.where(col <= row, s, -1e30)
        m, l, acc = upd(s, m, l, acc, vblk)
        o_ref[0, 0] = (acc / l).astype(jnp.bfloat16)


def _attn_call(lens_x, qt, kt, vt):
    grid_spec = pltpu.PrefetchScalarGridSpec(
        num_scalar_prefetch=1,
        grid=(_B, _NH, _NQ),
        in_specs=[
            pl.BlockSpec((1, 1, _BQ, _HD), lambda b, h, qi, L: (b, h, qi, 0)),
            pl.BlockSpec((1, 1, _S, _HD), lambda b, h, qi, L: (b, h, 0, 0)),
            pl.BlockSpec((1, 1, _S, _HD), lambda b, h, qi, L: (b, h, 0, 0)),
        ],
        out_specs=pl.BlockSpec((1, 1, _BQ, _HD),
                               lambda b, h, qi, L: (b, h, qi, 0)),
    )
    return pl.pallas_call(
        _attn_body,
        grid_spec=grid_spec,
        out_shape=jax.ShapeDtypeStruct((_B, _NH, _S, _HD), jnp.bfloat16),
        compiler_params=pltpu.CompilerParams(
            dimension_semantics=("parallel", "parallel", "arbitrary")),
        interpret=_INTERPRET,
    )(lens_x, qt, kt, vt)


def _mlp_body(lens_ref, a_ref, hs_ref, wo_ref, g2_ref, wg_ref, wu_ref, wd_ref,
              o_ref):
    b = pl.program_id(0)
    qi = pl.program_id(1)
    ln = lens_ref[b, 0]

    @pl.when(qi * _BQ < ln)
    def _():
        r2 = hs_ref[0] + jnp.dot(a_ref[0], wo_ref[...],
                                 preferred_element_type=jnp.float32)
        var = jnp.mean(r2 * r2, axis=-1, keepdims=True)
        xn = ((r2 * lax.rsqrt(var + _EPS)) * g2_ref[0]).astype(jnp.bfloat16)
        g = jnp.dot(xn, wg_ref[...], preferred_element_type=jnp.float32)
        u = jnp.dot(xn, wu_ref[...], preferred_element_type=jnp.float32)
        act = (g * jax.nn.sigmoid(g) * u).astype(jnp.bfloat16)
        o_ref[0] = r2 + jnp.dot(act, wd_ref[...],
                                preferred_element_type=jnp.float32)


def _mlp_call(lens_x, attn, hs_c, wo, g2, wg, wu, wd):
    grid_spec = pltpu.PrefetchScalarGridSpec(
        num_scalar_prefetch=1,
        grid=(_B, _NQ),
        in_specs=[
            pl.BlockSpec((1, _BQ, _H), lambda b, qi, L: (b, qi, 0)),
            pl.BlockSpec((1, _BQ, _H), lambda b, qi, L: (b, qi, 0)),
            pl.BlockSpec((_H, _H), lambda b, qi, L: (0, 0)),
            pl.BlockSpec((1, _H), lambda b, qi, L: (0, 0)),
            pl.BlockSpec((_H, _F), lambda b, qi, L: (0, 0)),
            pl.BlockSpec((_H, _F), lambda b, qi, L: (0, 0)),
            pl.BlockSpec((_F, _H), lambda b, qi, L: (0, 0)),
        ],
        out_specs=pl.BlockSpec((1, _BQ, _H), lambda b, qi, L: (b, qi, 0)),
    )
    return pl.pallas_call(
        _mlp_body,
        grid_spec=grid_spec,
        out_shape=jax.ShapeDtypeStruct((_B, _S, _H), jnp.float32),
        compiler_params=pltpu.CompilerParams(
            dimension_semantics=("parallel", "parallel")),
        interpret=_INTERPRET,
    )(lens_x, attn, hs_c, wo, g2, wg, wu, wd)


def kernel(hidden_states, position_ids, topk_mask, topk_scores, g1, g2,
           Wq, Wk, Wv, Wo, Wg, Wu, Wd):
    mask_i = topk_mask.astype(jnp.int32)
    gidx, lens_x = _sc_index_build(mask_i)

    hid_flat = hidden_states.reshape(_B * _S, _H)
    hs_c_flat = _sc_gather(hid_flat, gidx.reshape(-1))
    hs_c = hs_c_flat.reshape(_B, _S, _H)

    pos3 = gidx.reshape(_B * _NQ, _BQ, 1)
    wqkv = jnp.concatenate([Wq, Wk, Wv], axis=1).astype(jnp.bfloat16)
    q, k, v = _qkv_call(lens_x, hs_c, pos3, wqkv, g1.reshape(1, _H))

    def to4(x):
        return x.reshape(_B, _S, _NH, _HD).transpose(0, 2, 1, 3)

    attn4 = _attn_call(lens_x, to4(q), to4(k), to4(v))
    attn = attn4.transpose(0, 2, 1, 3).reshape(_B, _S, _H)

    layer_out = _mlp_call(lens_x, attn, hs_c,
                          Wo.astype(jnp.bfloat16), g2.reshape(1, _H),
                          Wg.astype(jnp.bfloat16), Wu.astype(jnp.bfloat16),
                          Wd.astype(jnp.bfloat16))

    outp = _sc_scatter(hid_flat, layer_out.reshape(_B * _S, _H),
                       mask_i.reshape(-1), gidx.reshape(-1))
    return outp[:_B * _S].reshape(_B, _S, _H)


# speedup vs baseline: 1.2221x; 1.2221x over previous
import functools

import numpy as np

import jax
import jax.numpy as jnp
from jax import lax
from jax.experimental import pallas as pl
from jax.experimental.pallas import tpu as pltpu
from jax.experimental.pallas import tpu_sc as plsc

_B, _S, _H, _NH, _HD, _F = 2, 4096, 1024, 16, 64, 2816
_EPS = 1e-5
_THETA = 10000.0
_BQ = 256
_BK = 512
_NQ = _S // _BQ
_TRASH = _B * _S
_NTILES = 32
_RPT = _B * _S // _NTILES
_SUB = 32
_NCH = _RPT // _SUB

_INTERPRET = False


def _sc_index_build(mask_i32):
    mesh = plsc.VectorSubcoreMesh(core_axis_name="c", subcore_axis_name="s", num_cores=2, num_subcores=16)

    @functools.partial(
        pl.kernel,
        out_type=(
            jax.ShapeDtypeStruct((_B, _S), jnp.int32),
            jax.ShapeDtypeStruct((_B, 16), jnp.int32),
        ),
        mesh=mesh,
        scratch_types=[
            pltpu.VMEM((_S,), jnp.int32),
            pltpu.VMEM((_S,), jnp.int32),
            pltpu.VMEM((16,), jnp.int32),
        ],
        compiler_params=pltpu.CompilerParams(needs_layout_passes=False),
        interpret=_INTERPRET,
    )
    def k(mask_hbm, gidx_hbm, lens_hbm, mask_v, gidx_v, lens_v):
        wid = lax.axis_index("s") * 2 + lax.axis_index("c")

        @pl.when(wid == 0)
        def _():
            def batch_body(b, _):
                pltpu.sync_copy(mask_hbm.at[b], mask_v)
                neg1 = jnp.full((16,), -1, jnp.int32)

                def initb(i, c):
                    gidx_v[pl.ds(i * 16, 16)] = neg1
                    return c

                lax.fori_loop(0, _S // 16, initb, 0)
                base = b * _S

                def chunk(i, carry):
                    m = mask_v[pl.ds(i * 16, 16)]
                    mb = m != 0
                    c = plsc.cumsum(m)
                    rank = c - 1 + carry
                    tvec = lax.iota(jnp.int32, 16) + i * 16 + base
                    plsc.store_scatter(gidx_v, [rank], tvec, mask=mb)
                    return carry + jnp.sum(m)

                ln = lax.fori_loop(0, _S // 16, chunk, jnp.int32(0))
                pltpu.sync_copy(gidx_v, gidx_hbm.at[b])
                lens_v[...] = jnp.zeros((16,), jnp.int32) + ln
                pltpu.sync_copy(lens_v, lens_hbm.at[b])
                return 0

            lax.fori_loop(0, _B, batch_body, 0)

    return k(mask_i32)


def _sc_gather(hid_flat, gidx_flat):
    mesh = plsc.VectorSubcoreMesh(core_axis_name="c", subcore_axis_name="s", num_cores=2, num_subcores=16)

    @functools.partial(
        pl.kernel,
        out_type=jax.ShapeDtypeStruct((_B * _S, _H), jnp.float32),
        mesh=mesh,
        scratch_types=[
            pltpu.VMEM((_RPT,), jnp.int32),
            pltpu.VMEM((_SUB, _H), jnp.float32),
            pltpu.VMEM((_SUB, _H), jnp.float32),
            pltpu.VMEM((_SUB, _H), jnp.float32),
            pltpu.SemaphoreType.DMA,
            pltpu.SemaphoreType.DMA,
            pltpu.SemaphoreType.DMA,
            pltpu.SemaphoreType.DMA,
            pltpu.SemaphoreType.DMA,
            pltpu.SemaphoreType.DMA,
        ],
        interpret=_INTERPRET,
    )
    def k(hid_hbm, gidx_hbm, out_hbm, idx_all, buf0, buf1, buf2,
          sg0, sg1, sg2, sw0, sw1, sw2):
        wid = lax.axis_index("s") * 2 + lax.axis_index("c")
        base = wid * _RPT
        bbase = (base // _S) * _S
        pltpu.sync_copy(gidx_hbm.at[pl.ds(base, _RPT)], idx_all)
        for t in range(_RPT // 16):
            g = idx_all[pl.ds(t * 16, 16)]
            idx_all[pl.ds(t * 16, 16)] = jnp.where(g < 0, bbase, g)
        bufs = (buf0, buf1, buf2)
        sgs = (sg0, sg1, sg2)
        sws = (sw0, sw1, sw2)
        gd = [None, None, None]
        wd = [None, None, None]

        def fire_g(j):
            gd[j % 3] = pltpu.async_copy(
                hid_hbm.at[idx_all.at[pl.ds(j * _SUB, _SUB)]],
                bufs[j % 3], sgs[j % 3])

        def fire_w(j):
            wd[j % 3] = pltpu.async_copy(
                bufs[j % 3], out_hbm.at[pl.ds(base + j * _SUB, _SUB)],
                sws[j % 3])

        for j in range(3):
            fire_g(j)
        for j in range(_NCH):
            gd[j % 3].wait()
            fire_w(j)
            if j + 3 < _NCH:
                wd[j % 3].wait()
                fire_g(j + 3)
        for j in range(_NCH - 3, _NCH):
            wd[j % 3].wait()

    return k(hid_flat, gidx_flat)


def _sc_scatter(hid_flat, lo_flat, mask_flat, gidx_flat):
    mesh = plsc.VectorSubcoreMesh(core_axis_name="c", subcore_axis_name="s", num_cores=2, num_subcores=16)

    @functools.partial(
        pl.kernel,
        out_type=jax.ShapeDtypeStruct((_B * _S + 8, _H), jnp.float32),
        mesh=mesh,
        scratch_types=[
            pltpu.VMEM((_RPT,), jnp.int32),
            pltpu.VMEM((_NCH, _SUB), jnp.int32),
            pltpu.VMEM((_NCH, _SUB), jnp.int32),
            pltpu.VMEM((_SUB, _H), jnp.float32),
            pltpu.VMEM((_SUB, _H), jnp.float32),
            pltpu.VMEM((_SUB, _H), jnp.float32),
            pltpu.SemaphoreType.DMA,
            pltpu.SemaphoreType.DMA,
            pltpu.SemaphoreType.DMA,
            pltpu.SemaphoreType.DMA,
            pltpu.SemaphoreType.DMA,
            pltpu.SemaphoreType.DMA,
        ],
        interpret=_INTERPRET,
    )
    def k(hid_hbm, lo_hbm, mask_hbm, gidx_hbm, out_hbm, tmp, idxA, idxB,
          buf0, buf1, buf2, sl0, sl1, sl2, ss0, ss1, ss2):
        wid = lax.axis_index("s") * 2 + lax.axis_index("c")
        base = wid * _RPT

        pltpu.sync_copy(mask_hbm.at[pl.ds(base, _RPT)], tmp)
        for t in range(_RPT // 16):
            m = tmp[pl.ds(t * 16, 16)]
            tvec = lax.iota(jnp.int32, 16) + (base + t * 16)
            idxA[t // 2, pl.ds((t % 2) * 16, 16)] = jnp.where(
                m != 0, _TRASH, tvec)
        pltpu.sync_copy(gidx_hbm.at[pl.ds(base, _RPT)], tmp)
        for t in range(_RPT // 16):
            g = tmp[pl.ds(t * 16, 16)]
            idxB[t // 2, pl.ds((t % 2) * 16, 16)] = jnp.where(
                g < 0, _TRASH, g)

        jobs = [(hid_hbm, idxA, j) for j in range(_NCH)] + \
               [(lo_hbm, idxB, j) for j in range(_NCH)]
        nj = len(jobs)
        bufs = (buf0, buf1, buf2)
        sls = (sl0, sl1, sl2)
        sss = (ss0, ss1, ss2)
        ld = [None, None, None]
        sd = [None, None, None]

        def fire_l(i):
            src, _, j = jobs[i]
            ld[i % 3] = pltpu.async_copy(
                src.at[pl.ds(base + j * _SUB, _SUB)], bufs[i % 3], sls[i % 3])

        def fire_s(i):
            _, idxr, j = jobs[i]
            sd[i % 3] = pltpu.async_copy(
                bufs[i % 3], out_hbm.at[idxr.at[j]], sss[i % 3])

        for i in range(3):
            fire_l(i)
        for i in range(nj):
            ld[i % 3].wait()
            fire_s(i)
            if i + 3 < nj:
                sd[i % 3].wait()
                fire_l(i + 3)
        for i in range(nj - 3, nj):
            sd[i % 3].wait()

    return k(hid_flat, lo_flat, mask_flat, gidx_flat)


def _qkv_body(lens_ref, hs_ref, pos_ref, w_ref, g_ref, q_ref, k_ref, v_ref):
    b = pl.program_id(0)
    qi = pl.program_id(1)
    ln = lens_ref[b, 0]

    @pl.when(qi * _BQ < ln)
    def _():
        x = hs_ref[0]
        var = jnp.mean(x * x, axis=-1, keepdims=True)
        xn = (x * lax.rsqrt(var + _EPS)) * g_ref[0]
        qkv = jnp.dot(xn.astype(jnp.bfloat16), w_ref[...],
                      preferred_element_type=jnp.float32)
        pos = pos_ref[0].astype(jnp.float32) - b * float(_S)
        l_idx = lax.broadcasted_iota(jnp.int32, (1, _H), 1)
        jmod = (l_idx % 32).astype(jnp.float32)
        invf = jnp.exp(jmod * (-np.log(_THETA) / 32.0))
        ang = pos * invf
        c = jnp.cos(ang)
        s = jnp.sin(ang)
        sel = (l_idx % 64) < 32

        def rope(t):
            xp = jnp.concatenate([t[:, 32:], t[:, :32]], axis=1)
            xm = jnp.concatenate([t[:, -32:], t[:, :-32]], axis=1)
            return jnp.where(sel, -xp, xm)

        qp = qkv[:, :_H]
        kp = qkv[:, _H:2 * _H]
        q_ref[0] = (qp * c + rope(qp) * s).astype(jnp.bfloat16)
        k_ref[0] = (kp * c + rope(kp) * s).astype(jnp.bfloat16)
        v_ref[0] = qkv[:, 2 * _H:].astype(jnp.bfloat16)

    @pl.when(qi * _BQ >= ln)
    def _():
        z = jnp.zeros((_BQ, _H), jnp.bfloat16)
        q_ref[0] = z
        k_ref[0] = z
        v_ref[0] = z


def _qkv_call(lens_x, hs_c, pos3, wqkv, g1):
    grid_spec = pltpu.PrefetchScalarGridSpec(
        num_scalar_prefetch=1,
        grid=(_B, _NQ),
        in_specs=[
            pl.BlockSpec((1, _BQ, _H), lambda b, qi, L: (b, qi, 0)),
            pl.BlockSpec((1, _BQ, 1), lambda b, qi, L: (b * _NQ + qi, 0, 0)),
            pl.BlockSpec((_H, 3 * _H), lambda b, qi, L: (0, 0)),
            pl.BlockSpec((1, _H), lambda b, qi, L: (0, 0)),
        ],
        out_specs=[
            pl.BlockSpec((1, _BQ, _H), lambda b, qi, L: (b, qi, 0)),
            pl.BlockSpec((1, _BQ, _H), lambda b, qi, L: (b, qi, 0)),
            pl.BlockSpec((1, _BQ, _H), lambda b, qi, L: (b, qi, 0)),
        ],
    )
    shp = jax.ShapeDtypeStruct((_B, _S, _H), jnp.bfloat16)
    return pl.pallas_call(
        _qkv_body,
        grid_spec=grid_spec,
        out_shape=[shp, shp, shp],
        compiler_params=pltpu.CompilerParams(
            dimension_semantics=("parallel", "parallel")),
        interpret=_INTERPRET,
    )(lens_x, hs_c, pos3, wqkv, g1)


def _attn_body(lens_ref, q_ref, k_ref, v_ref, o_ref, k0s, k1s, v0s, v1s):
    b = pl.program_id(0)
    qi = pl.program_id(2)
    start = qi * _BQ
    ln = lens_ref[b, 0]

    @pl.when(qi == 0)
    def _():
        k0s[...] = k_ref[0][:, :_HD]
        k1s[...] = k_ref[0][:, _HD:]
        v0s[...] = v_ref[0][:, :_HD]
        v1s[...] = v_ref[0][:, _HD:]

    @pl.when(start < ln)
    def _():
        qq = q_ref[0]
        q0 = qq[:, :_HD]
        q1 = qq[:, _HD:]
        scale = 1.0 / np.sqrt(_HD)

        def upd(s, m, l, acc, vblk):
            m_new = jnp.maximum(m, jnp.max(s, axis=1, keepdims=True))
            alpha = jnp.exp(m - m_new)
            p = jnp.exp(s - m_new)
            l_new = l * alpha + jnp.sum(p, axis=1, keepdims=True)
            acc_new = acc * alpha + jnp.dot(p.astype(jnp.bfloat16), vblk,
                                            preferred_element_type=jnp.float32)
            return m_new, l_new, acc_new

        def blockstep(kb, carry, masked):
            m0, l0, a0, m1, l1, a1 = carry
            kb0 = k0s[pl.ds(kb * _BK, _BK), :]
            kb1 = k1s[pl.ds(kb * _BK, _BK), :]
            vb0 = v0s[pl.ds(kb * _BK, _BK), :]
            vb1 = v1s[pl.ds(kb * _BK, _BK), :]
            s0 = lax.dot_general(q0, kb0, (((1,), (1,)), ((), ())),
                                 preferred_element_type=jnp.float32) * scale
            s1 = lax.dot_general(q1, kb1, (((1,), (1,)), ((), ())),
                                 preferred_element_type=jnp.float32) * scale
            if masked:
                row = start + lax.broadcasted_iota(jnp.int32, (_BQ, 1), 0)
                col = kb * _BK + lax.broadcasted_iota(jnp.int32, (1, _BK), 1)
                ok = col <= row
                s0 = jnp.where(ok, s0, -1e30)
                s1 = jnp.where(ok, s1, -1e30)
            m0, l0, a0 = upd(s0, m0, l0, a0, vb0)
            m1, l1, a1 = upd(s1, m1, l1, a1, vb1)
            return m0, l0, a0, m1, l1, a1

        ndiag = start // _BK
        mi = jnp.full((_BQ, 1), -1e30, jnp.float32)
        li = jnp.zeros((_BQ, 1), jnp.float32)
        ai = jnp.zeros((_BQ, _HD), jnp.float32)
        carry = lax.fori_loop(
            0, ndiag, lambda kb, c: blockstep(kb, c, False),
            (mi, li, ai, mi, li, ai))
        m0, l0, a0, m1, l1, a1 = blockstep(ndiag, carry, True)
        o_ref[0] = jnp.concatenate(
            [(a0 / l0), (a1 / l1)], axis=1).astype(jnp.bfloat16)


def _attn_call(lens_x, q, k, v):
    grid_spec = pltpu.PrefetchScalarGridSpec(
        num_scalar_prefetch=1,
        grid=(_B, _NH // 2, _NQ),
        in_specs=[
            pl.BlockSpec((1, _BQ, 2 * _HD), lambda b, h, qi, L: (b, qi, h)),
            pl.BlockSpec((1, _S, 2 * _HD), lambda b, h, qi, L: (b, 0, h)),
            pl.BlockSpec((1, _S, 2 * _HD), lambda b, h, qi, L: (b, 0, h)),
        ],
        out_specs=pl.BlockSpec((1, _BQ, 2 * _HD),
                               lambda b, h, qi, L: (b, qi, h)),
        scratch_shapes=[
            pltpu.VMEM((_S, _HD), jnp.bfloat16),
            pltpu.VMEM((_S, _HD), jnp.bfloat16),
            pltpu.VMEM((_S, _HD), jnp.bfloat16),
            pltpu.VMEM((_S, _HD), jnp.bfloat16),
        ],
    )
    return pl.pallas_call(
        _attn_body,
        grid_spec=grid_spec,
        out_shape=jax.ShapeDtypeStruct((_B, _S, _H), jnp.bfloat16),
        compiler_params=pltpu.CompilerParams(
            dimension_semantics=("parallel", "parallel", "arbitrary")),
        interpret=_INTERPRET,
    )(lens_x, q, k, v)


def _mlp_body(lens_ref, a_ref, hs_ref, wo_ref, g2_ref, wg_ref, wu_ref, wd_ref,
              o_ref):
    b = pl.program_id(0)
    qi = pl.program_id(1)
    ln = lens_ref[b, 0]

    @pl.when(qi * _BQ < ln)
    def _():
        r2 = hs_ref[0] + jnp.dot(a_ref[0], wo_ref[...],
                                 preferred_element_type=jnp.float32)
        var = jnp.mean(r2 * r2, axis=-1, keepdims=True)
        xn = ((r2 * lax.rsqrt(var + _EPS)) * g2_ref[0]).astype(jnp.bfloat16)
        g = jnp.dot(xn, wg_ref[...], preferred_element_type=jnp.float32)
        u = jnp.dot(xn, wu_ref[...], preferred_element_type=jnp.float32)
        act = (g * jax.nn.sigmoid(g) * u).astype(jnp.bfloat16)
        o_ref[0] = r2 + jnp.dot(act, wd_ref[...],
                                preferred_element_type=jnp.float32)


def _mlp_call(lens_x, attn, hs_c, wo, g2, wg, wu, wd):
    grid_spec = pltpu.PrefetchScalarGridSpec(
        num_scalar_prefetch=1,
        grid=(_B, _NQ),
        in_specs=[
            pl.BlockSpec((1, _BQ, _H), lambda b, qi, L: (b, qi, 0)),
            pl.BlockSpec((1, _BQ, _H), lambda b, qi, L: (b, qi, 0)),
            pl.BlockSpec((_H, _H), lambda b, qi, L: (0, 0)),
            pl.BlockSpec((1, _H), lambda b, qi, L: (0, 0)),
            pl.BlockSpec((_H, _F), lambda b, qi, L: (0, 0)),
            pl.BlockSpec((_H, _F), lambda b, qi, L: (0, 0)),
            pl.BlockSpec((_F, _H), lambda b, qi, L: (0, 0)),
        ],
        out_specs=pl.BlockSpec((1, _BQ, _H), lambda b, qi, L: (b, qi, 0)),
    )
    return pl.pallas_call(
        _mlp_body,
        grid_spec=grid_spec,
        out_shape=jax.ShapeDtypeStruct((_B, _S, _H), jnp.float32),
        compiler_params=pltpu.CompilerParams(
            dimension_semantics=("parallel", "parallel")),
        interpret=_INTERPRET,
    )(lens_x, attn, hs_c, wo, g2, wg, wu, wd)


def kernel(hidden_states, position_ids, topk_mask, topk_scores, g1, g2,
           Wq, Wk, Wv, Wo, Wg, Wu, Wd):
    mask_i = topk_mask.astype(jnp.int32)
    gidx, lens_x = _sc_index_build(mask_i)

    hid_flat = hidden_states.reshape(_B * _S, _H)
    hs_c_flat = _sc_gather(hid_flat, gidx.reshape(-1))
    hs_c = hs_c_flat.reshape(_B, _S, _H)

    pos3 = gidx.reshape(_B * _NQ, _BQ, 1)
    wqkv = jnp.concatenate([Wq, Wk, Wv], axis=1).astype(jnp.bfloat16)
    q, k, v = _qkv_call(lens_x, hs_c, pos3, wqkv, g1.reshape(1, _H))

    attn = _attn_call(lens_x, q, k, v)

    layer_out = _mlp_call(lens_x, attn, hs_c,
                          Wo.astype(jnp.bfloat16), g2.reshape(1, _H),
                          Wg.astype(jnp.bfloat16), Wu.astype(jnp.bfloat16),
                          Wd.astype(jnp.bfloat16))

    outp = _sc_scatter(hid_flat, layer_out.reshape(_B * _S, _H),
                       mask_i.reshape(-1), gidx.reshape(-1))
    return outp[:_B * _S].reshape(_B, _S, _H)


# speedup vs baseline: 1.2978x; 1.0619x over previous
import functools

import numpy as np

import jax
import jax.numpy as jnp
from jax import lax
from jax.experimental import pallas as pl
from jax.experimental.pallas import tpu as pltpu
from jax.experimental.pallas import tpu_sc as plsc

_B, _S, _H, _NH, _HD, _F = 2, 4096, 1024, 16, 64, 2816
_EPS = 1e-5
_THETA = 10000.0
_BQ = 256
_BK = 512
_NQ = _S // _BQ
_TRASH = _B * _S
_NTILES = 32
_RPT = _B * _S // _NTILES
_SUB = 32
_NCH = _RPT // _SUB

_INTERPRET = False


def _sc_index_build(mask_i32):
    mesh = plsc.VectorSubcoreMesh(core_axis_name="c", subcore_axis_name="s", num_cores=2, num_subcores=16)

    @functools.partial(
        pl.kernel,
        out_type=(
            jax.ShapeDtypeStruct((_B, _S), jnp.int32),
            jax.ShapeDtypeStruct((_B, 16), jnp.int32),
        ),
        mesh=mesh,
        scratch_types=[
            pltpu.VMEM((_S,), jnp.int32),
            pltpu.VMEM((_S,), jnp.int32),
            pltpu.VMEM((16,), jnp.int32),
        ],
        compiler_params=pltpu.CompilerParams(needs_layout_passes=False),
        interpret=_INTERPRET,
    )
    def k(mask_hbm, gidx_hbm, lens_hbm, mask_v, gidx_v, lens_v):
        wid = lax.axis_index("s") * 2 + lax.axis_index("c")

        @pl.when(wid == 0)
        def _():
            def batch_body(b, _):
                pltpu.sync_copy(mask_hbm.at[b], mask_v)
                neg1 = jnp.full((16,), -1, jnp.int32)

                def initb(i, c):
                    gidx_v[pl.ds(i * 16, 16)] = neg1
                    return c

                lax.fori_loop(0, _S // 16, initb, 0)
                base = b * _S

                def chunk(i, carry):
                    m = mask_v[pl.ds(i * 16, 16)]
                    mb = m != 0
                    c = plsc.cumsum(m)
                    rank = c - 1 + carry
                    tvec = lax.iota(jnp.int32, 16) + i * 16 + base
                    plsc.store_scatter(gidx_v, [rank], tvec, mask=mb)
                    return carry + jnp.sum(m)

                ln = lax.fori_loop(0, _S // 16, chunk, jnp.int32(0))
                pltpu.sync_copy(gidx_v, gidx_hbm.at[b])
                lens_v[...] = jnp.zeros((16,), jnp.int32) + ln
                pltpu.sync_copy(lens_v, lens_hbm.at[b])
                return 0

            lax.fori_loop(0, _B, batch_body, 0)

    return k(mask_i32)


def _sc_gather(hid_flat, gidx_flat):
    mesh = plsc.VectorSubcoreMesh(core_axis_name="c", subcore_axis_name="s", num_cores=2, num_subcores=16)

    @functools.partial(
        pl.kernel,
        out_type=jax.ShapeDtypeStruct((_B * _S, _H), jnp.float32),
        mesh=mesh,
        scratch_types=[
            pltpu.VMEM((_RPT,), jnp.int32),
            pltpu.VMEM((_SUB, _H), jnp.float32),
            pltpu.VMEM((_SUB, _H), jnp.float32),
            pltpu.VMEM((_SUB, _H), jnp.float32),
            pltpu.SemaphoreType.DMA,
            pltpu.SemaphoreType.DMA,
            pltpu.SemaphoreType.DMA,
            pltpu.SemaphoreType.DMA,
            pltpu.SemaphoreType.DMA,
            pltpu.SemaphoreType.DMA,
        ],
        interpret=_INTERPRET,
    )
    def k(hid_hbm, gidx_hbm, out_hbm, idx_all, buf0, buf1, buf2,
          sg0, sg1, sg2, sw0, sw1, sw2):
        wid = lax.axis_index("s") * 2 + lax.axis_index("c")
        base = wid * _RPT
        bbase = (base // _S) * _S
        pltpu.sync_copy(gidx_hbm.at[pl.ds(base, _RPT)], idx_all)
        for t in range(_RPT // 16):
            g = idx_all[pl.ds(t * 16, 16)]
            idx_all[pl.ds(t * 16, 16)] = jnp.where(g < 0, bbase, g)
        bufs = (buf0, buf1, buf2)
        sgs = (sg0, sg1, sg2)
        sws = (sw0, sw1, sw2)

        def g_desc(j):
            return pltpu.make_async_copy(
                hid_hbm.at[idx_all.at[pl.ds(j * _SUB, _SUB)]],
                bufs[j % 3], sgs[j % 3])

        def w_desc(j):
            return pltpu.make_async_copy(
                bufs[j % 3], out_hbm.at[pl.ds(base + j * _SUB, _SUB)],
                sws[j % 3])

        for j in range(3):
            g_desc(j).start()
        for j in range(_NCH):
            g_desc(j).wait()
            w_desc(j).start()
            if j + 3 < _NCH:
                w_desc(j).wait()
                g_desc(j + 3).start()
        for j in range(_NCH - 3, _NCH):
            w_desc(j).wait()

    return k(hid_flat, gidx_flat)


def _sc_scatter(hid_flat, lo_flat, gidx_flat):
    mesh = plsc.VectorSubcoreMesh(core_axis_name="c", subcore_axis_name="s", num_cores=2, num_subcores=16)

    @functools.partial(
        pl.kernel,
        out_type=jax.ShapeDtypeStruct((_B * _S + 8, _H), jnp.float32),
        mesh=mesh,
        scratch_types=[
            pltpu.VMEM((_S + _SUB,), jnp.int32),
            pltpu.VMEM((_SUB,), jnp.int32),
            pltpu.VMEM((_SUB,), jnp.int32),
            pltpu.VMEM((_SUB,), jnp.int32),
            pltpu.VMEM((_SUB, _H), jnp.float32),
            pltpu.VMEM((_SUB, _H), jnp.float32),
            pltpu.VMEM((_SUB, _H), jnp.float32),
            pltpu.SemaphoreType.DMA,
            pltpu.SemaphoreType.DMA,
            pltpu.SemaphoreType.DMA,
            pltpu.SemaphoreType.DMA,
            pltpu.SemaphoreType.DMA,
            pltpu.SemaphoreType.DMA,
        ],
        compiler_params=pltpu.CompilerParams(needs_layout_passes=False),
        interpret=_INTERPRET,
    )
    def k(hid_hbm, lo_hbm, gidx_hbm, out_hbm, gv, ib0, ib1, ib2,
          buf0, buf1, buf2, sl0, sl1, sl2, ss0, ss1, ss2):
        wid = lax.axis_index("s") * 2 + lax.axis_index("c")
        base = wid * _RPT
        bidx = base // _S
        bbase = bidx * _S
        bufs = (buf0, buf1, buf2)
        ibs = (ib0, ib1, ib2)
        sls = (sl0, sl1, sl2)
        sss = (ss0, ss1, ss2)

        def bl_desc(j):
            return pltpu.make_async_copy(
                hid_hbm.at[pl.ds(base + j * _SUB, _SUB)],
                bufs[j % 3], sls[j % 3])

        def bw_desc(j):
            return pltpu.make_async_copy(
                bufs[j % 3], out_hbm.at[pl.ds(base + j * _SUB, _SUB)],
                sss[j % 3])

        for j in range(3):
            bl_desc(j).start()
        for j in range(_NCH):
            bl_desc(j).wait()
            bw_desc(j).start()
            if j + 3 < _NCH:
                bw_desc(j).wait()
                bl_desc(j + 3).start()
        for j in range(_NCH - 3, _NCH):
            bw_desc(j).wait()

        pltpu.sync_copy(gidx_hbm.at[pl.ds(bbase, _S)], gv.at[pl.ds(0, _S)])

        def cnt(i, carry):
            lo, hi = carry
            g = gv[pl.ds(i * 16, 16)]
            ok = g >= 0
            lo = lo + jnp.sum((ok & (g < base)).astype(jnp.int32))
            hi = hi + jnp.sum((ok & (g < base + _RPT)).astype(jnp.int32))
            return lo, hi

        r_lo, r_hi = lax.fori_loop(0, _S // 16, cnt,
                                   (jnp.int32(0), jnp.int32(0)))
        r8 = (r_lo // 8) * 8

        _NJ = _NCH + 1

        def rs_of(j):
            return jnp.minimum(r8 + j * _SUB, _S - _SUB)

        def l_desc(j):
            return pltpu.make_async_copy(
                lo_hbm.at[pl.ds(bbase + rs_of(j), _SUB)],
                bufs[j % 3], sls[j % 3])

        def s_desc(j):
            return pltpu.make_async_copy(
                bufs[j % 3], out_hbm.at[ibs[j % 3]], sss[j % 3])

        def build_idx(j):
            rs = rs_of(j)
            for t in range(_SUB // 16):
                g = gv[pl.ds(rs + t * 16, 16)]
                lane_r = lax.iota(jnp.int32, 16) + (rs + t * 16)
                ibs[j % 3][pl.ds(t * 16, 16)] = jnp.where(
                    (g < 0) | (lane_r >= r_hi), _TRASH, g)

        for j in range(3):
            build_idx(j)
            l_desc(j).start()
        for j in range(_NJ):
            l_desc(j).wait()
            s_desc(j).start()
            if j + 3 < _NJ:
                s_desc(j).wait()
                build_idx(j + 3)
                l_desc(j + 3).start()
        for j in range(_NJ - 3, _NJ):
            s_desc(j).wait()

    return k(hid_flat, lo_flat, gidx_flat)


def _bcast_cols(col, width):
    ones = jnp.ones((1, width), jnp.float32)
    return lax.dot_general(col, ones, (((1,), (0,)), ((), ())),
                           preferred_element_type=jnp.float32)


def _qkv_body(lens_ref, hs_ref, pos_ref, w_ref, g_ref, q_ref, k_ref, v_ref):
    b = pl.program_id(0)
    qi = pl.program_id(1)
    ln = lens_ref[b, 0]

    @pl.when(qi * _BQ < ln)
    def _():
        x = hs_ref[0]
        var = jnp.mean(x * x, axis=-1, keepdims=True)
        rb = _bcast_cols(lax.rsqrt(var + _EPS), _H)
        xn = (x * rb) * g_ref[0]
        qkv = jnp.dot(xn.astype(jnp.bfloat16), w_ref[...],
                      preferred_element_type=jnp.float32)
        pos = pos_ref[0].astype(jnp.float32) - b * float(_S)
        l_idx = lax.broadcasted_iota(jnp.int32, (1, _H), 1)
        jmod = (l_idx % 32).astype(jnp.float32)
        invf = jnp.exp(jmod * (-np.log(_THETA) / 32.0))
        ang = lax.dot_general(pos, invf, (((1,), (0,)), ((), ())),
                              preferred_element_type=jnp.float32)
        c = jnp.cos(ang)
        s = jnp.sin(ang)
        sel = (l_idx % 64) < 32

        def rope(t):
            xp = jnp.concatenate([t[:, 32:], t[:, :32]], axis=1)
            xm = jnp.concatenate([t[:, -32:], t[:, :-32]], axis=1)
            return jnp.where(sel, -xp, xm)

        qp = qkv[:, :_H]
        kp = qkv[:, _H:2 * _H]
        q_ref[0] = (qp * c + rope(qp) * s).astype(jnp.bfloat16)
        k_ref[0] = (kp * c + rope(kp) * s).astype(jnp.bfloat16)
        v_ref[0] = qkv[:, 2 * _H:].astype(jnp.bfloat16)

    @pl.when(qi * _BQ >= ln)
    def _():
        z = jnp.zeros((_BQ, _H), jnp.bfloat16)
        q_ref[0] = z
        k_ref[0] = z
        v_ref[0] = z


def _qkv_call(lens_x, hs_c, pos3, wqkv, g1):
    grid_spec = pltpu.PrefetchScalarGridSpec(
        num_scalar_prefetch=1,
        grid=(_B, _NQ),
        in_specs=[
            pl.BlockSpec((1, _BQ, _H), lambda b, qi, L: (b, qi, 0)),
            pl.BlockSpec((1, _BQ, 1), lambda b, qi, L: (b * _NQ + qi, 0, 0)),
            pl.BlockSpec((_H, 3 * _H), lambda b, qi, L: (0, 0)),
            pl.BlockSpec((1, _H), lambda b, qi, L: (0, 0)),
        ],
        out_specs=[
            pl.BlockSpec((1, _BQ, _H), lambda b, qi, L: (b, qi, 0)),
            pl.BlockSpec((1, _BQ, _H), lambda b, qi, L: (b, qi, 0)),
            pl.BlockSpec((1, _BQ, _H), lambda b, qi, L: (b, qi, 0)),
        ],
    )
    shp = jax.ShapeDtypeStruct((_B, _S, _H), jnp.bfloat16)
    return pl.pallas_call(
        _qkv_body,
        grid_spec=grid_spec,
        out_shape=[shp, shp, shp],
        compiler_params=pltpu.CompilerParams(
            dimension_semantics=("parallel", "parallel")),
        interpret=_INTERPRET,
    )(lens_x, hs_c, pos3, wqkv, g1)


def _attn_body(lens_ref, q_ref, k_ref, v_ref, o_ref, k0s, k1s, v0s, v1s):
    b = pl.program_id(0)
    qi = pl.program_id(2)
    start = qi * _BQ
    ln = lens_ref[b, 0]

    @pl.when(qi == 0)
    def _():
        k0s[...] = k_ref[0][:, :_HD]
        k1s[...] = k_ref[0][:, _HD:]
        v0s[...] = v_ref[0][:, :_HD]
        v1s[...] = v_ref[0][:, _HD:]

    @pl.when(start < ln)
    def _():
        qq = q_ref[0]
        q0 = qq[:, :_HD]
        q1 = qq[:, _HD:]
        scale = 1.0 / np.sqrt(_HD)

        def upd(s, m, l, acc, vblk):
            m_new = jnp.maximum(m, jnp.max(s, axis=1, keepdims=True))
            alpha = jnp.exp(m - m_new)
            p = jnp.exp(s - _bcast_cols(m_new, _BK))
            l_new = l * alpha + jnp.sum(p, axis=1, keepdims=True)
            acc_new = acc * _bcast_cols(alpha, _HD) + jnp.dot(
                p.astype(jnp.bfloat16), vblk,
                preferred_element_type=jnp.float32)
            return m_new, l_new, acc_new

        def blockstep(kb, carry, masked):
            m0, l0, a0, m1, l1, a1 = carry
            kb0 = k0s[pl.ds(kb * _BK, _BK), :]
            kb1 = k1s[pl.ds(kb * _BK, _BK), :]
            vb0 = v0s[pl.ds(kb * _BK, _BK), :]
            vb1 = v1s[pl.ds(kb * _BK, _BK), :]
            s0 = lax.dot_general(q0, kb0, (((1,), (1,)), ((), ())),
                                 preferred_element_type=jnp.float32) * scale
            s1 = lax.dot_general(q1, kb1, (((1,), (1,)), ((), ())),
                                 preferred_element_type=jnp.float32) * scale
            if masked:
                row = start + lax.broadcasted_iota(jnp.int32, (_BQ, 1), 0)
                col = kb * _BK + lax.broadcasted_iota(jnp.int32, (1, _BK), 1)
                ok = col <= row
                s0 = jnp.where(ok, s0, -1e30)
                s1 = jnp.where(ok, s1, -1e30)
            m0, l0, a0 = upd(s0, m0, l0, a0, vb0)
            m1, l1, a1 = upd(s1, m1, l1, a1, vb1)
            return m0, l0, a0, m1, l1, a1

        ndiag = start // _BK
        mi = jnp.full((_BQ, 1), -1e30, jnp.float32)
        li = jnp.zeros((_BQ, 1), jnp.float32)
        ai = jnp.zeros((_BQ, _HD), jnp.float32)
        carry = lax.fori_loop(
            0, ndiag, lambda kb, c: blockstep(kb, c, False),
            (mi, li, ai, mi, li, ai))
        m0, l0, a0, m1, l1, a1 = blockstep(ndiag, carry, True)
        o_ref[0] = jnp.concatenate(
            [a0 * _bcast_cols(1.0 / l0, _HD),
             a1 * _bcast_cols(1.0 / l1, _HD)], axis=1).astype(jnp.bfloat16)


def _attn_call(lens_x, q, k, v):
    grid_spec = pltpu.PrefetchScalarGridSpec(
        num_scalar_prefetch=1,
        grid=(_B, _NH // 2, _NQ),
        in_specs=[
            pl.BlockSpec((1, _BQ, 2 * _HD), lambda b, h, qi, L: (b, qi, h)),
            pl.BlockSpec((1, _S, 2 * _HD), lambda b, h, qi, L: (b, 0, h)),
            pl.BlockSpec((1, _S, 2 * _HD), lambda b, h, qi, L: (b, 0, h)),
        ],
        out_specs=pl.BlockSpec((1, _BQ, 2 * _HD),
                               lambda b, h, qi, L: (b, qi, h)),
        scratch_shapes=[
            pltpu.VMEM((_S, _HD), jnp.bfloat16),
            pltpu.VMEM((_S, _HD), jnp.bfloat16),
            pltpu.VMEM((_S, _HD), jnp.bfloat16),
            pltpu.VMEM((_S, _HD), jnp.bfloat16),
        ],
    )
    return pl.pallas_call(
        _attn_body,
        grid_spec=grid_spec,
        out_shape=jax.ShapeDtypeStruct((_B, _S, _H), jnp.bfloat16),
        compiler_params=pltpu.CompilerParams(
            dimension_semantics=("parallel", "parallel", "arbitrary")),
        interpret=_INTERPRET,
    )(lens_x, q, k, v)


def _mlp_body(lens_ref, a_ref, hs_ref, wo_ref, g2_ref, wg_ref, wu_ref, wd_ref,
              o_ref):
    b = pl.program_id(0)
    qi = pl.program_id(1)
    ln = lens_ref[b, 0]

    @pl.when(qi * _BQ < ln)
    def _():
        r2 = hs_ref[0] + jnp.dot(a_ref[0], wo_ref[...],
                                 preferred_element_type=jnp.float32)
        var = jnp.mean(r2 * r2, axis=-1, keepdims=True)
        rb = _bcast_cols(lax.rsqrt(var + _EPS), _H)
        xn = ((r2 * rb) * g2_ref[0]).astype(jnp.bfloat16)
        g = jnp.dot(xn, wg_ref[...], preferred_element_type=jnp.float32)
        u = jnp.dot(xn, wu_ref[...], preferred_element_type=jnp.float32)
        act = (g * jax.nn.sigmoid(g) * u).astype(jnp.bfloat16)
        o_ref[0] = r2 + jnp.dot(act, wd_ref[...],
                                preferred_element_type=jnp.float32)


def _mlp_call(lens_x, attn, hs_c, wo, g2, wg, wu, wd):
    grid_spec = pltpu.PrefetchScalarGridSpec(
        num_scalar_prefetch=1,
        grid=(_B, _NQ),
        in_specs=[
            pl.BlockSpec((1, _BQ, _H), lambda b, qi, L: (b, qi, 0)),
            pl.BlockSpec((1, _BQ, _H), lambda b, qi, L: (b, qi, 0)),
            pl.BlockSpec((_H, _H), lambda b, qi, L: (0, 0)),
            pl.BlockSpec((1, _H), lambda b, qi, L: (0, 0)),
            pl.BlockSpec((_H, _F), lambda b, qi, L: (0, 0)),
            pl.BlockSpec((_H, _F), lambda b, qi, L: (0, 0)),
            pl.BlockSpec((_F, _H), lambda b, qi, L: (0, 0)),
        ],
        out_specs=pl.BlockSpec((1, _BQ, _H), lambda b, qi, L: (b, qi, 0)),
    )
    return pl.pallas_call(
        _mlp_body,
        grid_spec=grid_spec,
        out_shape=jax.ShapeDtypeStruct((_B, _S, _H), jnp.float32),
        compiler_params=pltpu.CompilerParams(
            dimension_semantics=("parallel", "parallel")),
        interpret=_INTERPRET,
    )(lens_x, attn, hs_c, wo, g2, wg, wu, wd)


def kernel(hidden_states, position_ids, topk_mask, topk_scores, g1, g2,
           Wq, Wk, Wv, Wo, Wg, Wu, Wd):
    mask_i = topk_mask.astype(jnp.int32)
    gidx, lens_x = _sc_index_build(mask_i)

    hid_flat = hidden_states.reshape(_B * _S, _H)
    hs_c_flat = _sc_gather(hid_flat, gidx.reshape(-1))
    hs_c = hs_c_flat.reshape(_B, _S, _H)

    pos3 = gidx.reshape(_B * _NQ, _BQ, 1)
    wqkv = jnp.concatenate([Wq, Wk, Wv], axis=1).astype(jnp.bfloat16)
    q, k, v = _qkv_call(lens_x, hs_c, pos3, wqkv, g1.reshape(1, _H))

    attn = _attn_call(lens_x, q, k, v)

    layer_out = _mlp_call(lens_x, attn, hs_c,
                          Wo.astype(jnp.bfloat16), g2.reshape(1, _H),
                          Wg.astype(jnp.bfloat16), Wu.astype(jnp.bfloat16),
                          Wd.astype(jnp.bfloat16))

    outp = _sc_scatter(hid_flat, layer_out.reshape(_B * _S, _H),
                       gidx.reshape(-1))
    return outp[:_B * _S].reshape(_B, _S, _H)


# speedup vs baseline: 1.5604x; 1.2023x over previous
import functools

import numpy as np

import jax
import jax.numpy as jnp
from jax import lax
from jax.experimental import pallas as pl
from jax.experimental.pallas import tpu as pltpu
from jax.experimental.pallas import tpu_sc as plsc

_B, _S, _H, _NH, _HD, _F = 2, 4096, 1024, 16, 64, 2816
_EPS = 1e-5
_THETA = 10000.0
_BQ = 512
_BK = 512
_NQ = _S // _BQ
_TRASH = _B * _S
_NTILES = 32
_RPT = _B * _S // _NTILES
_SUB = 32
_NCH = _RPT // _SUB

_INTERPRET = False


def _sc_index_build(mask_i32):
    mesh = plsc.VectorSubcoreMesh(core_axis_name="c", subcore_axis_name="s", num_cores=2, num_subcores=16)

    @functools.partial(
        pl.kernel,
        out_type=(
            jax.ShapeDtypeStruct((_B, _S), jnp.int32),
            jax.ShapeDtypeStruct((_B, 16), jnp.int32),
        ),
        mesh=mesh,
        scratch_types=[
            pltpu.VMEM((_S,), jnp.int32),
            pltpu.VMEM((_S,), jnp.int32),
            pltpu.VMEM((16,), jnp.int32),
        ],
        compiler_params=pltpu.CompilerParams(needs_layout_passes=False),
        interpret=_INTERPRET,
    )
    def k(mask_hbm, gidx_hbm, lens_hbm, mask_v, gidx_v, lens_v):
        wid = lax.axis_index("s") * 2 + lax.axis_index("c")

        @pl.when(wid == 0)
        def _():
            def batch_body(b, _):
                pltpu.sync_copy(mask_hbm.at[b], mask_v)
                neg1 = jnp.full((16,), -1, jnp.int32)

                def initb(i, c):
                    gidx_v[pl.ds(i * 16, 16)] = neg1
                    return c

                lax.fori_loop(0, _S // 16, initb, 0)
                base = b * _S

                def chunk(i, carry):
                    m = mask_v[pl.ds(i * 16, 16)]
                    mb = m != 0
                    c = plsc.cumsum(m)
                    rank = c - 1 + carry
                    tvec = lax.iota(jnp.int32, 16) + i * 16 + base
                    plsc.store_scatter(gidx_v, [rank], tvec, mask=mb)
                    return carry + jnp.sum(m)

                ln = lax.fori_loop(0, _S // 16, chunk, jnp.int32(0))
                pltpu.sync_copy(gidx_v, gidx_hbm.at[b])
                lens_v[...] = jnp.zeros((16,), jnp.int32) + ln
                pltpu.sync_copy(lens_v, lens_hbm.at[b])
                return 0

            lax.fori_loop(0, _B, batch_body, 0)

    return k(mask_i32)


def _sc_gather(hid_flat, gidx_flat):
    mesh = plsc.VectorSubcoreMesh(core_axis_name="c", subcore_axis_name="s", num_cores=2, num_subcores=16)

    @functools.partial(
        pl.kernel,
        out_type=jax.ShapeDtypeStruct((_B * _S, _H), jnp.float32),
        mesh=mesh,
        scratch_types=[
            pltpu.VMEM((_RPT,), jnp.int32),
            pltpu.VMEM((_SUB, _H), jnp.float32),
            pltpu.VMEM((_SUB, _H), jnp.float32),
            pltpu.VMEM((_SUB, _H), jnp.float32),
            pltpu.SemaphoreType.DMA,
            pltpu.SemaphoreType.DMA,
            pltpu.SemaphoreType.DMA,
            pltpu.SemaphoreType.DMA,
            pltpu.SemaphoreType.DMA,
            pltpu.SemaphoreType.DMA,
        ],
        interpret=_INTERPRET,
    )
    def k(hid_hbm, gidx_hbm, out_hbm, idx_all, buf0, buf1, buf2,
          sg0, sg1, sg2, sw0, sw1, sw2):
        wid = lax.axis_index("s") * 2 + lax.axis_index("c")
        base = wid * _RPT
        bbase = (base // _S) * _S
        pltpu.sync_copy(gidx_hbm.at[pl.ds(base, _RPT)], idx_all)
        for t in range(_RPT // 16):
            g = idx_all[pl.ds(t * 16, 16)]
            idx_all[pl.ds(t * 16, 16)] = jnp.where(g < 0, bbase, g)
        bufs = (buf0, buf1, buf2)
        sgs = (sg0, sg1, sg2)
        sws = (sw0, sw1, sw2)

        def g_desc(j):
            return pltpu.make_async_copy(
                hid_hbm.at[idx_all.at[pl.ds(j * _SUB, _SUB)]],
                bufs[j % 3], sgs[j % 3])

        def w_desc(j):
            return pltpu.make_async_copy(
                bufs[j % 3], out_hbm.at[pl.ds(base + j * _SUB, _SUB)],
                sws[j % 3])

        for j in range(3):
            g_desc(j).start()
        for j in range(_NCH):
            g_desc(j).wait()
            w_desc(j).start()
            if j + 3 < _NCH:
                w_desc(j).wait()
                g_desc(j + 3).start()
        for j in range(_NCH - 3, _NCH):
            w_desc(j).wait()

    return k(hid_flat, gidx_flat)


def _sc_scatter(hid_flat, lo_flat, gidx_flat):
    mesh = plsc.VectorSubcoreMesh(core_axis_name="c", subcore_axis_name="s", num_cores=2, num_subcores=16)

    @functools.partial(
        pl.kernel,
        out_type=jax.ShapeDtypeStruct((_B * _S + 8, _H), jnp.float32),
        mesh=mesh,
        scratch_types=[
            pltpu.VMEM((_S + _SUB,), jnp.int32),
            pltpu.VMEM((_SUB,), jnp.int32),
            pltpu.VMEM((_SUB,), jnp.int32),
            pltpu.VMEM((_SUB,), jnp.int32),
            pltpu.VMEM((_SUB, _H), jnp.float32),
            pltpu.VMEM((_SUB, _H), jnp.float32),
            pltpu.VMEM((_SUB, _H), jnp.float32),
            pltpu.SemaphoreType.DMA,
            pltpu.SemaphoreType.DMA,
            pltpu.SemaphoreType.DMA,
            pltpu.SemaphoreType.DMA,
            pltpu.SemaphoreType.DMA,
            pltpu.SemaphoreType.DMA,
        ],
        compiler_params=pltpu.CompilerParams(needs_layout_passes=False),
        interpret=_INTERPRET,
    )
    def k(hid_hbm, lo_hbm, gidx_hbm, out_hbm, gv, ib0, ib1, ib2,
          buf0, buf1, buf2, sl0, sl1, sl2, ss0, ss1, ss2):
        wid = lax.axis_index("s") * 2 + lax.axis_index("c")
        base = wid * _RPT
        bidx = base // _S
        bbase = bidx * _S
        bufs = (buf0, buf1, buf2)
        ibs = (ib0, ib1, ib2)
        sls = (sl0, sl1, sl2)
        sss = (ss0, ss1, ss2)

        def bl_desc(j):
            return pltpu.make_async_copy(
                hid_hbm.at[pl.ds(base + j * _SUB, _SUB)],
                bufs[j % 3], sls[j % 3])

        def bw_desc(j):
            return pltpu.make_async_copy(
                bufs[j % 3], out_hbm.at[pl.ds(base + j * _SUB, _SUB)],
                sss[j % 3])

        for j in range(3):
            bl_desc(j).start()
        for j in range(_NCH):
            bl_desc(j).wait()
            bw_desc(j).start()
            if j + 3 < _NCH:
                bw_desc(j).wait()
                bl_desc(j + 3).start()
        for j in range(_NCH - 3, _NCH):
            bw_desc(j).wait()

        pltpu.sync_copy(gidx_hbm.at[pl.ds(bbase, _S)], gv.at[pl.ds(0, _S)])

        def cnt(i, carry):
            lo, hi = carry
            g = gv[pl.ds(i * 16, 16)]
            ok = g >= 0
            lo = lo + jnp.sum((ok & (g < base)).astype(jnp.int32))
            hi = hi + jnp.sum((ok & (g < base + _RPT)).astype(jnp.int32))
            return lo, hi

        r_lo, r_hi = lax.fori_loop(0, _S // 16, cnt,
                                   (jnp.int32(0), jnp.int32(0)))
        r8 = (r_lo // 8) * 8

        _NJ = _NCH + 1

        def rs_of(j):
            return jnp.minimum(r8 + j * _SUB, _S - _SUB)

        def l_desc(j):
            return pltpu.make_async_copy(
                lo_hbm.at[pl.ds(bbase + rs_of(j), _SUB)],
                bufs[j % 3], sls[j % 3])

        def s_desc(j):
            return pltpu.make_async_copy(
                bufs[j % 3], out_hbm.at[ibs[j % 3]], sss[j % 3])

        def build_idx(j):
            rs = rs_of(j)
            for t in range(_SUB // 16):
                g = gv[pl.ds(rs + t * 16, 16)]
                lane_r = lax.iota(jnp.int32, 16) + (rs + t * 16)
                ibs[j % 3][pl.ds(t * 16, 16)] = jnp.where(
                    (g < 0) | (lane_r >= r_hi), _TRASH, g)

        for j in range(3):
            build_idx(j)
            l_desc(j).start()
        for j in range(_NJ):
            l_desc(j).wait()
            s_desc(j).start()
            if j + 3 < _NJ:
                s_desc(j).wait()
                build_idx(j + 3)
                l_desc(j + 3).start()
        for j in range(_NJ - 3, _NJ):
            s_desc(j).wait()

    return k(hid_flat, lo_flat, gidx_flat)


def _tile_lanes(x, width):
    t = x
    while t.shape[1] < width:
        t = jnp.concatenate([t, t], axis=1)
    return t


def _qkv_body(lens_ref, hs_ref, pos_ref, w_ref, g_ref, q_ref, k_ref, v_ref):
    b = pl.program_id(0)
    qi = pl.program_id(1)
    ln = lens_ref[b, 0]

    @pl.when(qi * _BQ < ln)
    def _():
        x = hs_ref[0]
        var = jnp.mean(x * x, axis=-1, keepdims=True)
        xn = (x * lax.rsqrt(var + _EPS)) * g_ref[0]
        qkv = jnp.dot(xn.astype(jnp.bfloat16), w_ref[...],
                      preferred_element_type=jnp.float32)
        pos = pos_ref[0].astype(jnp.float32) - b * float(_S)
        j32 = lax.broadcasted_iota(jnp.int32, (1, 32), 1).astype(jnp.float32)
        invf = jnp.exp(j32 * (-np.log(_THETA) / 32.0))
        ang = pos * invf
        c = _tile_lanes(jnp.cos(ang), _H)
        s = _tile_lanes(jnp.sin(ang), _H)
        l_idx = lax.broadcasted_iota(jnp.int32, (1, _H), 1)
        sel = (l_idx % 64) < 32

        def rope(t):
            xp = jnp.concatenate([t[:, 32:], t[:, :32]], axis=1)
            xm = jnp.concatenate([t[:, -32:], t[:, :-32]], axis=1)
            return jnp.where(sel, -xp, xm)

        qp = qkv[:, :_H]
        kp = qkv[:, _H:2 * _H]
        q_ref[0] = (qp * c + rope(qp) * s).astype(jnp.bfloat16)
        k_ref[0] = (kp * c + rope(kp) * s).astype(jnp.bfloat16)
        v_ref[0] = qkv[:, 2 * _H:].astype(jnp.bfloat16)

    @pl.when(qi * _BQ >= ln)
    def _():
        z = jnp.zeros((_BQ, _H), jnp.bfloat16)
        q_ref[0] = z
        k_ref[0] = z
        v_ref[0] = z


def _qkv_call(lens_x, hs_c, pos3, wqkv, g1):
    grid_spec = pltpu.PrefetchScalarGridSpec(
        num_scalar_prefetch=1,
        grid=(_B, _NQ),
        in_specs=[
            pl.BlockSpec((1, _BQ, _H), lambda b, qi, L: (b, qi, 0)),
            pl.BlockSpec((1, _BQ, 1), lambda b, qi, L: (b * _NQ + qi, 0, 0)),
            pl.BlockSpec((_H, 3 * _H), lambda b, qi, L: (0, 0)),
            pl.BlockSpec((1, _H), lambda b, qi, L: (0, 0)),
        ],
        out_specs=[
            pl.BlockSpec((1, _BQ, _H), lambda b, qi, L: (b, qi, 0)),
            pl.BlockSpec((1, _BQ, _H), lambda b, qi, L: (b, qi, 0)),
            pl.BlockSpec((1, _BQ, _H), lambda b, qi, L: (b, qi, 0)),
        ],
    )
    shp = jax.ShapeDtypeStruct((_B, _S, _H), jnp.bfloat16)
    return pl.pallas_call(
        _qkv_body,
        grid_spec=grid_spec,
        out_shape=[shp, shp, shp],
        compiler_params=pltpu.CompilerParams(
            dimension_semantics=("parallel", "parallel")),
        interpret=_INTERPRET,
    )(lens_x, hs_c, pos3, wqkv, g1)


def _attn_body(lens_ref, q_ref, k_ref, v_ref, o_ref, k0s, k1s, v0s, v1s):
    b = pl.program_id(0)
    qi = pl.program_id(2)
    start = qi * _BQ
    ln = lens_ref[b, 0]

    @pl.when(qi == 0)
    def _():
        k0s[...] = k_ref[0][:, :_HD]
        k1s[...] = k_ref[0][:, _HD:]
        v0s[...] = v_ref[0][:, :_HD]
        v1s[...] = v_ref[0][:, _HD:]

    @pl.when(start < ln)
    def _():
        qq = q_ref[0]
        q0 = qq[:, :_HD]
        q1 = qq[:, _HD:]
        scale = 1.0 / np.sqrt(_HD)

        def upd(s, m, l, acc, vblk):
            m_new = jnp.maximum(m, jnp.max(s, axis=1, keepdims=True))
            alpha = jnp.exp(m - m_new)
            p = jnp.exp(s - m_new)
            l_new = l * alpha + jnp.sum(p, axis=1, keepdims=True)
            acc_new = acc * alpha + jnp.dot(p.astype(jnp.bfloat16), vblk,
                                            preferred_element_type=jnp.float32)
            return m_new, l_new, acc_new

        def blockstep(kb, carry, masked):
            m0, l0, a0, m1, l1, a1 = carry
            kb0 = k0s[pl.ds(kb * _BK, _BK), :]
            kb1 = k1s[pl.ds(kb * _BK, _BK), :]
            vb0 = v0s[pl.ds(kb * _BK, _BK), :]
            vb1 = v1s[pl.ds(kb * _BK, _BK), :]
            s0 = lax.dot_general(q0, kb0, (((1,), (1,)), ((), ())),
                                 preferred_element_type=jnp.float32) * scale
            s1 = lax.dot_general(q1, kb1, (((1,), (1,)), ((), ())),
                                 preferred_element_type=jnp.float32) * scale
            if masked:
                row = start + lax.broadcasted_iota(jnp.int32, (_BQ, 1), 0)
                col = kb * _BK + lax.broadcasted_iota(jnp.int32, (1, _BK), 1)
                ok = col <= row
                s0 = jnp.where(ok, s0, -1e30)
                s1 = jnp.where(ok, s1, -1e30)
            m0, l0, a0 = upd(s0, m0, l0, a0, vb0)
            m1, l1, a1 = upd(s1, m1, l1, a1, vb1)
            return m0, l0, a0, m1, l1, a1

        ndiag = start // _BK
        mi = jnp.full((_BQ, 1), -1e30, jnp.float32)
        li = jnp.zeros((_BQ, 1), jnp.float32)
        ai = jnp.zeros((_BQ, _HD), jnp.float32)
        carry = lax.fori_loop(
            0, ndiag, lambda kb, c: blockstep(kb, c, False),
            (mi, li, ai, mi, li, ai))
        m0, l0, a0, m1, l1, a1 = blockstep(ndiag, carry, True)
        o_ref[0] = jnp.concatenate(
            [(a0 / l0), (a1 / l1)], axis=1).astype(jnp.bfloat16)


def _attn_call(lens_x, q, k, v):
    grid_spec = pltpu.PrefetchScalarGridSpec(
        num_scalar_prefetch=1,
        grid=(_B, _NH // 2, _NQ),
        in_specs=[
            pl.BlockSpec((1, _BQ, 2 * _HD), lambda b, h, qi, L: (b, qi, h)),
            pl.BlockSpec((1, _S, 2 * _HD), lambda b, h, qi, L: (b, 0, h)),
            pl.BlockSpec((1, _S, 2 * _HD), lambda b, h, qi, L: (b, 0, h)),
        ],
        out_specs=pl.BlockSpec((1, _BQ, 2 * _HD),
                               lambda b, h, qi, L: (b, qi, h)),
        scratch_shapes=[
            pltpu.VMEM((_S, _HD), jnp.bfloat16),
            pltpu.VMEM((_S, _HD), jnp.bfloat16),
            pltpu.VMEM((_S, _HD), jnp.bfloat16),
            pltpu.VMEM((_S, _HD), jnp.bfloat16),
        ],
    )
    return pl.pallas_call(
        _attn_body,
        grid_spec=grid_spec,
        out_shape=jax.ShapeDtypeStruct((_B, _S, _H), jnp.bfloat16),
        compiler_params=pltpu.CompilerParams(
            dimension_semantics=("parallel", "parallel", "arbitrary")),
        interpret=_INTERPRET,
    )(lens_x, q, k, v)


def _mlp_body(lens_ref, a_ref, hs_ref, wo_ref, g2_ref, wg_ref, wu_ref, wd_ref,
              o_ref):
    b = pl.program_id(0)
    qi = pl.program_id(1)
    ln = lens_ref[b, 0]

    @pl.when(qi * _BQ < ln)
    def _():
        r2 = hs_ref[0] + jnp.dot(a_ref[0], wo_ref[...],
                                 preferred_element_type=jnp.float32)
        var = jnp.mean(r2 * r2, axis=-1, keepdims=True)
        xn = ((r2 * lax.rsqrt(var + _EPS)) * g2_ref[0]).astype(jnp.bfloat16)
        g = jnp.dot(xn, wg_ref[...], preferred_element_type=jnp.float32)
        u = jnp.dot(xn, wu_ref[...], preferred_element_type=jnp.float32)
        act = (g * jax.nn.sigmoid(g) * u).astype(jnp.bfloat16)
        o_ref[0] = r2 + jnp.dot(act, wd_ref[...],
                                preferred_element_type=jnp.float32)


def _mlp_call(lens_x, attn, hs_c, wo, g2, wg, wu, wd):
    grid_spec = pltpu.PrefetchScalarGridSpec(
        num_scalar_prefetch=1,
        grid=(_B, _NQ),
        in_specs=[
            pl.BlockSpec((1, _BQ, _H), lambda b, qi, L: (b, qi, 0)),
            pl.BlockSpec((1, _BQ, _H), lambda b, qi, L: (b, qi, 0)),
            pl.BlockSpec((_H, _H), lambda b, qi, L: (0, 0)),
            pl.BlockSpec((1, _H), lambda b, qi, L: (0, 0)),
            pl.BlockSpec((_H, _F), lambda b, qi, L: (0, 0)),
            pl.BlockSpec((_H, _F), lambda b, qi, L: (0, 0)),
            pl.BlockSpec((_F, _H), lambda b, qi, L: (0, 0)),
        ],
        out_specs=pl.BlockSpec((1, _BQ, _H), lambda b, qi, L: (b, qi, 0)),
    )
    return pl.pallas_call(
        _mlp_body,
        grid_spec=grid_spec,
        out_shape=jax.ShapeDtypeStruct((_B, _S, _H), jnp.float32),
        compiler_params=pltpu.CompilerParams(
            dimension_semantics=("parallel", "parallel")),
        interpret=_INTERPRET,
    )(lens_x, attn, hs_c, wo, g2, wg, wu, wd)


def kernel(hidden_states, position_ids, topk_mask, topk_scores, g1, g2,
           Wq, Wk, Wv, Wo, Wg, Wu, Wd):
    mask_i = topk_mask.astype(jnp.int32)
    gidx, lens_x = _sc_index_build(mask_i)

    hid_flat = hidden_states.reshape(_B * _S, _H)
    hs_c_flat = _sc_gather(hid_flat, gidx.reshape(-1))
    hs_c = hs_c_flat.reshape(_B, _S, _H)

    pos3 = gidx.reshape(_B * _NQ, _BQ, 1)
    wqkv = jnp.concatenate([Wq, Wk, Wv], axis=1).astype(jnp.bfloat16)
    q, k, v = _qkv_call(lens_x, hs_c, pos3, wqkv, g1.reshape(1, _H))

    attn = _attn_call(lens_x, q, k, v)

    layer_out = _mlp_call(lens_x, attn, hs_c,
                          Wo.astype(jnp.bfloat16), g2.reshape(1, _H),
                          Wg.astype(jnp.bfloat16), Wu.astype(jnp.bfloat16),
                          Wd.astype(jnp.bfloat16))

    outp = _sc_scatter(hid_flat, layer_out.reshape(_B * _S, _H),
                       gidx.reshape(-1))
    return outp[:_B * _S].reshape(_B, _S, _H)


# speedup vs baseline: 2.1835x; 1.3994x over previous
import functools

import numpy as np

import jax
import jax.numpy as jnp
from jax import lax
from jax.experimental import pallas as pl
from jax.experimental.pallas import tpu as pltpu
from jax.experimental.pallas import tpu_sc as plsc

_B, _S, _H, _NH, _HD, _F = 2, 4096, 1024, 16, 64, 2816
_EPS = 1e-5
_THETA = 10000.0
_BQ = 512
_BK = 512
_NQ = _S // _BQ
_TRASH = _B * _S
_NTILES = 32
_RPT = _B * _S // _NTILES
_SUB = 32
_NCH = _RPT // _SUB

_INTERPRET = False


def _sc_index_build(mask_i32):
    mesh = plsc.VectorSubcoreMesh(core_axis_name="c", subcore_axis_name="s", num_cores=2, num_subcores=16)

    @functools.partial(
        pl.kernel,
        out_type=(
            jax.ShapeDtypeStruct((_B, _S), jnp.int32),
            jax.ShapeDtypeStruct((_B, 16), jnp.int32),
        ),
        mesh=mesh,
        scratch_types=[
            pltpu.VMEM((_S,), jnp.int32),
            pltpu.VMEM((_S,), jnp.int32),
            pltpu.VMEM((16,), jnp.int32),
        ],
        compiler_params=pltpu.CompilerParams(needs_layout_passes=False),
        interpret=_INTERPRET,
    )
    def k(mask_hbm, gidx_hbm, lens_hbm, mask_v, gidx_v, lens_v):
        wid = lax.axis_index("s") * 2 + lax.axis_index("c")

        @pl.when(wid == 0)
        def _():
            def batch_body(b, _):
                pltpu.sync_copy(mask_hbm.at[b], mask_v)
                neg1 = jnp.full((16,), -1, jnp.int32)

                def initb(i, c):
                    gidx_v[pl.ds(i * 16, 16)] = neg1
                    return c

                lax.fori_loop(0, _S // 16, initb, 0)
                base = b * _S

                def chunk(i, carry):
                    m = mask_v[pl.ds(i * 16, 16)]
                    mb = m != 0
                    c = plsc.cumsum(m)
                    rank = c - 1 + carry
                    tvec = lax.iota(jnp.int32, 16) + i * 16 + base
                    plsc.store_scatter(gidx_v, [rank], tvec, mask=mb)
                    return carry + jnp.sum(m)

                ln = lax.fori_loop(0, _S // 16, chunk, jnp.int32(0))
                pltpu.sync_copy(gidx_v, gidx_hbm.at[b])
                lens_v[...] = jnp.zeros((16,), jnp.int32) + ln
                pltpu.sync_copy(lens_v, lens_hbm.at[b])
                return 0

            lax.fori_loop(0, _B, batch_body, 0)

    return k(mask_i32)


def _sc_gather(hid_flat, gidx_flat):
    mesh = plsc.VectorSubcoreMesh(core_axis_name="c", subcore_axis_name="s", num_cores=2, num_subcores=16)

    @functools.partial(
        pl.kernel,
        out_type=jax.ShapeDtypeStruct((_B * _S, _H), jnp.float32),
        mesh=mesh,
        scratch_types=[
            pltpu.VMEM((_RPT,), jnp.int32),
            pltpu.VMEM((_SUB, _H), jnp.float32),
            pltpu.VMEM((_SUB, _H), jnp.float32),
            pltpu.VMEM((_SUB, _H), jnp.float32),
            pltpu.SemaphoreType.DMA,
            pltpu.SemaphoreType.DMA,
            pltpu.SemaphoreType.DMA,
            pltpu.SemaphoreType.DMA,
            pltpu.SemaphoreType.DMA,
            pltpu.SemaphoreType.DMA,
        ],
        compiler_params=pltpu.CompilerParams(needs_layout_passes=False),
        interpret=_INTERPRET,
    )
    def k(hid_hbm, gidx_hbm, out_hbm, idx_all, buf0, buf1, buf2,
          sg0, sg1, sg2, sw0, sw1, sw2):
        wid = lax.axis_index("s") * 2 + lax.axis_index("c")
        base = wid * _RPT
        bbase = (base // _S) * _S
        pltpu.sync_copy(gidx_hbm.at[pl.ds(base, _RPT)], idx_all)
        n = jnp.int32(0)
        for t in range(_RPT // 16):
            g = idx_all[pl.ds(t * 16, 16)]
            n = n + jnp.sum((g >= 0).astype(jnp.int32))
            idx_all[pl.ds(t * 16, 16)] = jnp.where(g < 0, bbase, g)
        bufs = (buf0, buf1, buf2)
        sgs = (sg0, sg1, sg2)
        sws = (sw0, sw1, sw2)

        def g_desc(j):
            return pltpu.make_async_copy(
                hid_hbm.at[idx_all.at[pl.ds(j * _SUB, _SUB)]],
                bufs[j % 3], sgs[j % 3])

        def w_desc(j):
            return pltpu.make_async_copy(
                bufs[j % 3], out_hbm.at[pl.ds(base + j * _SUB, _SUB)],
                sws[j % 3])

        for j in range(3):
            @pl.when(j * _SUB < n)
            def _(j=j):
                g_desc(j).start()
        for j in range(_NCH):
            @pl.when(j * _SUB < n)
            def _(j=j):
                g_desc(j).wait()
                w_desc(j).start()
            if j + 3 < _NCH:
                @pl.when((j + 3) * _SUB < n)
                def _(j=j):
                    w_desc(j).wait()
                    g_desc(j + 3).start()
        for j in range(_NCH):
            if j + 3 < _NCH:
                tail = (j * _SUB < n) & ((j + 3) * _SUB >= n)
            else:
                tail = j * _SUB < n

            @pl.when(tail)
            def _(j=j):
                w_desc(j).wait()

    return k(hid_flat, gidx_flat)


def _sc_scatter(hid_flat, lo_flat, gidx_flat):
    mesh = plsc.VectorSubcoreMesh(core_axis_name="c", subcore_axis_name="s", num_cores=2, num_subcores=16)

    @functools.partial(
        pl.kernel,
        out_type=jax.ShapeDtypeStruct((_B * _S + 8, _H), jnp.float32),
        mesh=mesh,
        scratch_types=[
            pltpu.VMEM((_S + _SUB,), jnp.int32),
            pltpu.VMEM((_SUB,), jnp.int32),
            pltpu.VMEM((_SUB,), jnp.int32),
            pltpu.VMEM((_SUB,), jnp.int32),
            pltpu.VMEM((_SUB, _H), jnp.float32),
            pltpu.VMEM((_SUB, _H), jnp.float32),
            pltpu.VMEM((_SUB, _H), jnp.float32),
            pltpu.SemaphoreType.DMA,
            pltpu.SemaphoreType.DMA,
            pltpu.SemaphoreType.DMA,
            pltpu.SemaphoreType.DMA,
            pltpu.SemaphoreType.DMA,
            pltpu.SemaphoreType.DMA,
        ],
        compiler_params=pltpu.CompilerParams(needs_layout_passes=False),
        interpret=_INTERPRET,
    )
    def k(hid_hbm, lo_hbm, gidx_hbm, out_hbm, gv, ib0, ib1, ib2,
          buf0, buf1, buf2, sl0, sl1, sl2, ss0, ss1, ss2):
        wid = lax.axis_index("s") * 2 + lax.axis_index("c")
        base = wid * _RPT
        bidx = base // _S
        bbase = bidx * _S
        bufs = (buf0, buf1, buf2)
        ibs = (ib0, ib1, ib2)
        sls = (sl0, sl1, sl2)
        sss = (ss0, ss1, ss2)

        def bl_desc(j):
            return pltpu.make_async_copy(
                hid_hbm.at[pl.ds(base + j * _SUB, _SUB)],
                bufs[j % 3], sls[j % 3])

        def bw_desc(j):
            return pltpu.make_async_copy(
                bufs[j % 3], out_hbm.at[pl.ds(base + j * _SUB, _SUB)],
                sss[j % 3])

        for j in range(3):
            bl_desc(j).start()
        for j in range(_NCH):
            bl_desc(j).wait()
            bw_desc(j).start()
            if j + 3 < _NCH:
                bw_desc(j).wait()
                bl_desc(j + 3).start()
        for j in range(_NCH - 3, _NCH):
            bw_desc(j).wait()

        pltpu.sync_copy(gidx_hbm.at[pl.ds(bbase, _S)], gv.at[pl.ds(0, _S)])

        def cnt(i, carry):
            lo, hi = carry
            g = gv[pl.ds(i * 16, 16)]
            ok = g >= 0
            lo = lo + jnp.sum((ok & (g < base)).astype(jnp.int32))
            hi = hi + jnp.sum((ok & (g < base + _RPT)).astype(jnp.int32))
            return lo, hi

        r_lo, r_hi = lax.fori_loop(0, _S // 16, cnt,
                                   (jnp.int32(0), jnp.int32(0)))
        r8 = (r_lo // 8) * 8

        _NJ = _NCH + 1

        def rs_of(j):
            return jnp.minimum(r8 + j * _SUB, _S - _SUB)

        def l_desc(j):
            return pltpu.make_async_copy(
                lo_hbm.at[pl.ds(bbase + rs_of(j), _SUB)],
                bufs[j % 3], sls[j % 3])

        def s_desc(j):
            return pltpu.make_async_copy(
                bufs[j % 3], out_hbm.at[ibs[j % 3]], sss[j % 3])

        def build_idx(j):
            rs = rs_of(j)
            for t in range(_SUB // 16):
                g = gv[pl.ds(rs + t * 16, 16)]
                lane_r = lax.iota(jnp.int32, 16) + (rs + t * 16)
                ibs[j % 3][pl.ds(t * 16, 16)] = jnp.where(
                    (g < 0) | (lane_r >= r_hi), _TRASH, g)

        def act(j):
            return r8 + j * _SUB < r_hi

        for j in range(3):
            @pl.when(act(j))
            def _(j=j):
                build_idx(j)
                l_desc(j).start()
        for j in range(_NJ):
            @pl.when(act(j))
            def _(j=j):
                l_desc(j).wait()
                s_desc(j).start()
            if j + 3 < _NJ:
                @pl.when(act(j + 3))
                def _(j=j):
                    s_desc(j).wait()
                    build_idx(j + 3)
                    l_desc(j + 3).start()
        for j in range(_NJ):
            if j + 3 < _NJ:
                tail = act(j) & jnp.logical_not(act(j + 3))
            else:
                tail = act(j)

            @pl.when(tail)
            def _(j=j):
                s_desc(j).wait()

    return k(hid_flat, lo_flat, gidx_flat)


def _tile_lanes(x, width):
    t = x
    while t.shape[1] < width:
        t = jnp.concatenate([t, t], axis=1)
    return t


def _qkv_body(lens_ref, hs_ref, pos_ref, w_ref, g_ref, q_ref, k_ref, v_ref):
    b = pl.program_id(0)
    qi = pl.program_id(1)
    ln = lens_ref[b, 0]

    @pl.when(qi * _BQ < ln)
    def _():
        x = hs_ref[0]
        var = jnp.mean(x * x, axis=-1, keepdims=True)
        xn = (x * lax.rsqrt(var + _EPS)) * g_ref[0]
        qkv = jnp.dot(xn.astype(jnp.bfloat16), w_ref[...],
                      preferred_element_type=jnp.float32)
        pos = pos_ref[0].astype(jnp.float32) - b * float(_S)
        j32 = lax.broadcasted_iota(jnp.int32, (1, 32), 1).astype(jnp.float32)
        invf = jnp.exp(j32 * (-np.log(_THETA) / 32.0))
        ang = pos * invf
        c = _tile_lanes(jnp.cos(ang), _H)
        s = _tile_lanes(jnp.sin(ang), _H)
        l_idx = lax.broadcasted_iota(jnp.int32, (1, _H), 1)
        sel = (l_idx % 64) < 32

        def rope(t):
            xp = jnp.concatenate([t[:, 32:], t[:, :32]], axis=1)
            xm = jnp.concatenate([t[:, -32:], t[:, :-32]], axis=1)
            return jnp.where(sel, -xp, xm)

        qp = qkv[:, :_H]
        kp = qkv[:, _H:2 * _H]
        q_ref[0] = (qp * c + rope(qp) * s).astype(jnp.bfloat16)
        k_ref[0] = (kp * c + rope(kp) * s).astype(jnp.bfloat16)
        v_ref[0] = qkv[:, 2 * _H:].astype(jnp.bfloat16)

    @pl.when(qi * _BQ >= ln)
    def _():
        z = jnp.zeros((_BQ, _H), jnp.bfloat16)
        q_ref[0] = z
        k_ref[0] = z
        v_ref[0] = z


def _qkv_call(lens_x, hs_c, pos3, wqkv, g1):
    grid_spec = pltpu.PrefetchScalarGridSpec(
        num_scalar_prefetch=1,
        grid=(_B, _NQ),
        in_specs=[
            pl.BlockSpec((1, _BQ, _H), lambda b, qi, L: (b, qi, 0)),
            pl.BlockSpec((1, _BQ, 1), lambda b, qi, L: (b * _NQ + qi, 0, 0)),
            pl.BlockSpec((_H, 3 * _H), lambda b, qi, L: (0, 0)),
            pl.BlockSpec((1, _H), lambda b, qi, L: (0, 0)),
        ],
        out_specs=[
            pl.BlockSpec((1, _BQ, _H), lambda b, qi, L: (b, qi, 0)),
            pl.BlockSpec((1, _BQ, _H), lambda b, qi, L: (b, qi, 0)),
            pl.BlockSpec((1, _BQ, _H), lambda b, qi, L: (b, qi, 0)),
        ],
    )
    shp = jax.ShapeDtypeStruct((_B, _S, _H), jnp.bfloat16)
    return pl.pallas_call(
        _qkv_body,
        grid_spec=grid_spec,
        out_shape=[shp, shp, shp],
        compiler_params=pltpu.CompilerParams(
            dimension_semantics=("parallel", "parallel")),
        interpret=_INTERPRET,
    )(lens_x, hs_c, pos3, wqkv, g1)


def _attn_body(lens_ref, q_ref, k_ref, v_ref, o_ref, k0s, k1s, v0s, v1s):
    b = pl.program_id(0)
    qi = pl.program_id(2)
    start = qi * _BQ
    ln = lens_ref[b, 0]

    @pl.when(qi == 0)
    def _():
        k0s[...] = k_ref[0][:, :_HD]
        k1s[...] = k_ref[0][:, _HD:]
        v0s[...] = v_ref[0][:, :_HD]
        v1s[...] = v_ref[0][:, _HD:]

    @pl.when(start < ln)
    def _():
        qq = q_ref[0]
        q0 = qq[:, :_HD]
        q1 = qq[:, _HD:]
        scale = 1.0 / np.sqrt(_HD)

        def upd(s, m, l, acc, vblk):
            m_new = jnp.maximum(m, jnp.max(s, axis=1, keepdims=True))
            alpha = jnp.exp(m - m_new)
            p = jnp.exp(s - m_new)
            l_new = l * alpha + jnp.sum(p, axis=1, keepdims=True)
            acc_new = acc * alpha + jnp.dot(p.astype(jnp.bfloat16), vblk,
                                            preferred_element_type=jnp.float32)
            return m_new, l_new, acc_new

        def blockstep(kb, carry, masked):
            m0, l0, a0, m1, l1, a1 = carry
            kb0 = k0s[pl.ds(kb * _BK, _BK), :]
            kb1 = k1s[pl.ds(kb * _BK, _BK), :]
            vb0 = v0s[pl.ds(kb * _BK, _BK), :]
            vb1 = v1s[pl.ds(kb * _BK, _BK), :]
            s0 = lax.dot_general(q0, kb0, (((1,), (1,)), ((), ())),
                                 preferred_element_type=jnp.float32) * scale
            s1 = lax.dot_general(q1, kb1, (((1,), (1,)), ((), ())),
                                 preferred_element_type=jnp.float32) * scale
            if masked:
                row = start + lax.broadcasted_iota(jnp.int32, (_BQ, 1), 0)
                col = kb * _BK + lax.broadcasted_iota(jnp.int32, (1, _BK), 1)
                ok = col <= row
                s0 = jnp.where(ok, s0, -1e30)
                s1 = jnp.where(ok, s1, -1e30)
            m0, l0, a0 = upd(s0, m0, l0, a0, vb0)
            m1, l1, a1 = upd(s1, m1, l1, a1, vb1)
            return m0, l0, a0, m1, l1, a1

        ndiag = start // _BK
        mi = jnp.full((_BQ, 1), -1e30, jnp.float32)
        li = jnp.zeros((_BQ, 1), jnp.float32)
        ai = jnp.zeros((_BQ, _HD), jnp.float32)
        carry = lax.fori_loop(
            0, ndiag, lambda kb, c: blockstep(kb, c, False),
            (mi, li, ai, mi, li, ai))
        m0, l0, a0, m1, l1, a1 = blockstep(ndiag, carry, True)
        o_ref[0] = jnp.concatenate(
            [(a0 / l0), (a1 / l1)], axis=1).astype(jnp.bfloat16)


def _attn_call(lens_x, q, k, v):
    grid_spec = pltpu.PrefetchScalarGridSpec(
        num_scalar_prefetch=1,
        grid=(_B, _NH // 2, _NQ),
        in_specs=[
            pl.BlockSpec((1, _BQ, 2 * _HD), lambda b, h, qi, L: (b, qi, h)),
            pl.BlockSpec((1, _S, 2 * _HD), lambda b, h, qi, L: (b, 0, h)),
            pl.BlockSpec((1, _S, 2 * _HD), lambda b, h, qi, L: (b, 0, h)),
        ],
        out_specs=pl.BlockSpec((1, _BQ, 2 * _HD),
                               lambda b, h, qi, L: (b, qi, h)),
        scratch_shapes=[
            pltpu.VMEM((_S, _HD), jnp.bfloat16),
            pltpu.VMEM((_S, _HD), jnp.bfloat16),
            pltpu.VMEM((_S, _HD), jnp.bfloat16),
            pltpu.VMEM((_S, _HD), jnp.bfloat16),
        ],
    )
    return pl.pallas_call(
        _attn_body,
        grid_spec=grid_spec,
        out_shape=jax.ShapeDtypeStruct((_B, _S, _H), jnp.bfloat16),
        compiler_params=pltpu.CompilerParams(
            dimension_semantics=("parallel", "parallel", "arbitrary")),
        interpret=_INTERPRET,
    )(lens_x, q, k, v)


def _mlp_body(lens_ref, a_ref, hs_ref, wo_ref, g2_ref, wg_ref, wu_ref, wd_ref,
              o_ref):
    b = pl.program_id(0)
    qi = pl.program_id(1)
    ln = lens_ref[b, 0]

    @pl.when(qi * _BQ < ln)
    def _():
        r2 = hs_ref[0] + jnp.dot(a_ref[0], wo_ref[...],
                                 preferred_element_type=jnp.float32)
        var = jnp.mean(r2 * r2, axis=-1, keepdims=True)
        xn = ((r2 * lax.rsqrt(var + _EPS)) * g2_ref[0]).astype(jnp.bfloat16)
        g = jnp.dot(xn, wg_ref[...], preferred_element_type=jnp.float32)
        u = jnp.dot(xn, wu_ref[...], preferred_element_type=jnp.float32)
        act = (g * jax.nn.sigmoid(g) * u).astype(jnp.bfloat16)
        o_ref[0] = r2 + jnp.dot(act, wd_ref[...],
                                preferred_element_type=jnp.float32)


def _mlp_call(lens_x, attn, hs_c, wo, g2, wg, wu, wd):
    grid_spec = pltpu.PrefetchScalarGridSpec(
        num_scalar_prefetch=1,
        grid=(_B, _NQ),
        in_specs=[
            pl.BlockSpec((1, _BQ, _H), lambda b, qi, L: (b, qi, 0)),
            pl.BlockSpec((1, _BQ, _H), lambda b, qi, L: (b, qi, 0)),
            pl.BlockSpec((_H, _H), lambda b, qi, L: (0, 0)),
            pl.BlockSpec((1, _H), lambda b, qi, L: (0, 0)),
            pl.BlockSpec((_H, _F), lambda b, qi, L: (0, 0)),
            pl.BlockSpec((_H, _F), lambda b, qi, L: (0, 0)),
            pl.BlockSpec((_F, _H), lambda b, qi, L: (0, 0)),
        ],
        out_specs=pl.BlockSpec((1, _BQ, _H), lambda b, qi, L: (b, qi, 0)),
    )
    return pl.pallas_call(
        _mlp_body,
        grid_spec=grid_spec,
        out_shape=jax.ShapeDtypeStruct((_B, _S, _H), jnp.float32),
        compiler_params=pltpu.CompilerParams(
            dimension_semantics=("parallel", "parallel")),
        interpret=_INTERPRET,
    )(lens_x, attn, hs_c, wo, g2, wg, wu, wd)


def kernel(hidden_states, position_ids, topk_mask, topk_scores, g1, g2,
           Wq, Wk, Wv, Wo, Wg, Wu, Wd):
    mask_i = topk_mask.astype(jnp.int32)
    gidx, lens_x = _sc_index_build(mask_i)

    hid_flat = hidden_states.reshape(_B * _S, _H)
    hs_c_flat = _sc_gather(hid_flat, gidx.reshape(-1))
    hs_c = hs_c_flat.reshape(_B, _S, _H)

    pos3 = gidx.reshape(_B * _NQ, _BQ, 1)
    wqkv = jnp.concatenate([Wq, Wk, Wv], axis=1).astype(jnp.bfloat16)
    q, k, v = _qkv_call(lens_x, hs_c, pos3, wqkv, g1.reshape(1, _H))

    attn = _attn_call(lens_x, q, k, v)

    layer_out = _mlp_call(lens_x, attn, hs_c,
                          Wo.astype(jnp.bfloat16), g2.reshape(1, _H),
                          Wg.astype(jnp.bfloat16), Wu.astype(jnp.bfloat16),
                          Wd.astype(jnp.bfloat16))

    outp = _sc_scatter(hid_flat, layer_out.reshape(_B * _S, _H),
                       gidx.reshape(-1))
    return outp[:_B * _S].reshape(_B, _S, _H)


# speedup vs baseline: 2.6056x; 1.1933x over previous
import functools

import numpy as np

import jax
import jax.numpy as jnp
from jax import lax
from jax.experimental import pallas as pl
from jax.experimental.pallas import tpu as pltpu
from jax.experimental.pallas import tpu_sc as plsc

_B, _S, _H, _NH, _HD, _F = 2, 4096, 1024, 16, 64, 2816
_EPS = 1e-5
_THETA = 10000.0
_BQ = 512
_BK = 512
_NQ = _S // _BQ
_TRASH = _B * _S
_NTILES = 32
_RPT = _B * _S // _NTILES
_SUB = 32
_NCH = _RPT // _SUB

_INTERPRET = False


def _sc_index_build(mask_i32):
    mesh = plsc.VectorSubcoreMesh(core_axis_name="c", subcore_axis_name="s", num_cores=2, num_subcores=16)

    @functools.partial(
        pl.kernel,
        out_type=(
            jax.ShapeDtypeStruct((_B, _S), jnp.int32),
            jax.ShapeDtypeStruct((_B, 16), jnp.int32),
        ),
        mesh=mesh,
        scratch_types=[
            pltpu.VMEM((_S,), jnp.int32),
            pltpu.VMEM((_S,), jnp.int32),
            pltpu.VMEM((16,), jnp.int32),
        ],
        compiler_params=pltpu.CompilerParams(needs_layout_passes=False),
        interpret=_INTERPRET,
    )
    def k(mask_hbm, gidx_hbm, lens_hbm, mask_v, gidx_v, lens_v):
        wid = lax.axis_index("s") * 2 + lax.axis_index("c")

        @pl.when(wid == 0)
        def _():
            def batch_body(b, _):
                pltpu.sync_copy(mask_hbm.at[b], mask_v)
                neg1 = jnp.full((16,), -1, jnp.int32)

                def initb(i, c):
                    gidx_v[pl.ds(i * 16, 16)] = neg1
                    return c

                lax.fori_loop(0, _S // 16, initb, 0)
                base = b * _S

                def chunk(i, carry):
                    m = mask_v[pl.ds(i * 16, 16)]
                    mb = m != 0
                    c = plsc.cumsum(m)
                    rank = c - 1 + carry
                    tvec = lax.iota(jnp.int32, 16) + i * 16 + base
                    plsc.store_scatter(gidx_v, [rank], tvec, mask=mb)
                    return carry + jnp.sum(m)

                ln = lax.fori_loop(0, _S // 16, chunk, jnp.int32(0))
                pltpu.sync_copy(gidx_v, gidx_hbm.at[b])
                lens_v[...] = jnp.zeros((16,), jnp.int32) + ln
                pltpu.sync_copy(lens_v, lens_hbm.at[b])
                return 0

            lax.fori_loop(0, _B, batch_body, 0)

    return k(mask_i32)


def _sc_gather(hid_flat, gidx_flat):
    mesh = plsc.VectorSubcoreMesh(core_axis_name="c", subcore_axis_name="s", num_cores=2, num_subcores=16)

    @functools.partial(
        pl.kernel,
        out_type=jax.ShapeDtypeStruct((_B * _S, _H), jnp.float32),
        mesh=mesh,
        scratch_types=[
            pltpu.VMEM((_RPT,), jnp.int32),
            pltpu.VMEM((_SUB, _H), jnp.float32),
            pltpu.VMEM((_SUB, _H), jnp.float32),
            pltpu.VMEM((_SUB, _H), jnp.float32),
            pltpu.SemaphoreType.DMA,
            pltpu.SemaphoreType.DMA,
            pltpu.SemaphoreType.DMA,
            pltpu.SemaphoreType.DMA,
            pltpu.SemaphoreType.DMA,
            pltpu.SemaphoreType.DMA,
        ],
        compiler_params=pltpu.CompilerParams(needs_layout_passes=False),
        interpret=_INTERPRET,
    )
    def k(hid_hbm, gidx_hbm, out_hbm, idx_all, buf0, buf1, buf2,
          sg0, sg1, sg2, sw0, sw1, sw2):
        wid = lax.axis_index("s") * 2 + lax.axis_index("c")
        base = wid * _RPT
        bbase = (base // _S) * _S
        pltpu.sync_copy(gidx_hbm.at[pl.ds(base, _RPT)], idx_all)
        n = jnp.int32(0)
        for t in range(_RPT // 16):
            g = idx_all[pl.ds(t * 16, 16)]
            n = n + jnp.sum((g >= 0).astype(jnp.int32))
            idx_all[pl.ds(t * 16, 16)] = jnp.where(g < 0, bbase, g)
        bufs = (buf0, buf1, buf2)
        sgs = (sg0, sg1, sg2)
        sws = (sw0, sw1, sw2)

        def g_desc(j):
            return pltpu.make_async_copy(
                hid_hbm.at[idx_all.at[pl.ds(j * _SUB, _SUB)]],
                bufs[j % 3], sgs[j % 3])

        def w_desc(j):
            return pltpu.make_async_copy(
                bufs[j % 3], out_hbm.at[pl.ds(base + j * _SUB, _SUB)],
                sws[j % 3])

        for j in range(3):
            @pl.when(j * _SUB < n)
            def _(j=j):
                g_desc(j).start()
        for j in range(_NCH):
            @pl.when(j * _SUB < n)
            def _(j=j):
                g_desc(j).wait()
                w_desc(j).start()
            if j + 3 < _NCH:
                @pl.when((j + 3) * _SUB < n)
                def _(j=j):
                    w_desc(j).wait()
                    g_desc(j + 3).start()
        for j in range(_NCH):
            if j + 3 < _NCH:
                tail = (j * _SUB < n) & ((j + 3) * _SUB >= n)
            else:
                tail = j * _SUB < n

            @pl.when(tail)
            def _(j=j):
                w_desc(j).wait()

    return k(hid_flat, gidx_flat)


def _sc_scatter(hid_flat, lo_flat, gidx_flat):
    mesh = plsc.VectorSubcoreMesh(core_axis_name="c", subcore_axis_name="s", num_cores=2, num_subcores=16)

    @functools.partial(
        pl.kernel,
        out_type=jax.ShapeDtypeStruct((_B * _S + 8, _H), jnp.float32),
        mesh=mesh,
        scratch_types=[
            pltpu.VMEM((_S + _SUB,), jnp.int32),
            pltpu.VMEM((_SUB,), jnp.int32),
            pltpu.VMEM((_SUB,), jnp.int32),
            pltpu.VMEM((_SUB,), jnp.int32),
            pltpu.VMEM((_SUB, _H), jnp.float32),
            pltpu.VMEM((_SUB, _H), jnp.float32),
            pltpu.VMEM((_SUB, _H), jnp.float32),
            pltpu.SemaphoreType.DMA,
            pltpu.SemaphoreType.DMA,
            pltpu.SemaphoreType.DMA,
            pltpu.SemaphoreType.DMA,
            pltpu.SemaphoreType.DMA,
            pltpu.SemaphoreType.DMA,
        ],
        compiler_params=pltpu.CompilerParams(needs_layout_passes=False),
        interpret=_INTERPRET,
    )
    def k(hid_hbm, lo_hbm, gidx_hbm, out_hbm, gv, ib0, ib1, ib2,
          buf0, buf1, buf2, sl0, sl1, sl2, ss0, ss1, ss2):
        wid = lax.axis_index("s") * 2 + lax.axis_index("c")
        base = wid * _RPT
        bidx = base // _S
        bbase = bidx * _S
        bufs = (buf0, buf1, buf2)
        ibs = (ib0, ib1, ib2)
        sls = (sl0, sl1, sl2)
        sss = (ss0, ss1, ss2)

        def bl_desc(j):
            return pltpu.make_async_copy(
                hid_hbm.at[pl.ds(base + j * _SUB, _SUB)],
                bufs[j % 3], sls[j % 3])

        def bw_desc(j):
            return pltpu.make_async_copy(
                bufs[j % 3], out_hbm.at[pl.ds(base + j * _SUB, _SUB)],
                sss[j % 3])

        for j in range(3):
            bl_desc(j).start()
        for j in range(_NCH):
            bl_desc(j).wait()
            bw_desc(j).start()
            if j + 3 < _NCH:
                bw_desc(j).wait()
                bl_desc(j + 3).start()
        for j in range(_NCH - 3, _NCH):
            bw_desc(j).wait()

        pltpu.sync_copy(gidx_hbm.at[pl.ds(bbase, _S)], gv.at[pl.ds(0, _S)])

        def cnt(i, carry):
            lo, hi = carry
            g = gv[pl.ds(i * 16, 16)]
            ok = g >= 0
            lo = lo + jnp.sum((ok & (g < base)).astype(jnp.int32))
            hi = hi + jnp.sum((ok & (g < base + _RPT)).astype(jnp.int32))
            return lo, hi

        r_lo, r_hi = lax.fori_loop(0, _S // 16, cnt,
                                   (jnp.int32(0), jnp.int32(0)))
        r8 = (r_lo // 8) * 8

        _NJ = _NCH + 1

        def rs_of(j):
            return jnp.minimum(r8 + j * _SUB, _S - _SUB)

        def l_desc(j):
            return pltpu.make_async_copy(
                lo_hbm.at[pl.ds(bbase + rs_of(j), _SUB)],
                bufs[j % 3], sls[j % 3])

        def s_desc(j):
            return pltpu.make_async_copy(
                bufs[j % 3], out_hbm.at[ibs[j % 3]], sss[j % 3])

        def build_idx(j):
            rs = rs_of(j)
            for t in range(_SUB // 16):
                g = gv[pl.ds(rs + t * 16, 16)]
                lane_r = lax.iota(jnp.int32, 16) + (rs + t * 16)
                ibs[j % 3][pl.ds(t * 16, 16)] = jnp.where(
                    (g < 0) | (lane_r >= r_hi), _TRASH, g)

        def act(j):
            return r8 + j * _SUB < r_hi

        for j in range(3):
            @pl.when(act(j))
            def _(j=j):
                build_idx(j)
                l_desc(j).start()
        for j in range(_NJ):
            @pl.when(act(j))
            def _(j=j):
                l_desc(j).wait()
                s_desc(j).start()
            if j + 3 < _NJ:
                @pl.when(act(j + 3))
                def _(j=j):
                    s_desc(j).wait()
                    build_idx(j + 3)
                    l_desc(j + 3).start()
        for j in range(_NJ):
            if j + 3 < _NJ:
                tail = act(j) & jnp.logical_not(act(j + 3))
            else:
                tail = act(j)

            @pl.when(tail)
            def _(j=j):
                s_desc(j).wait()

    return k(hid_flat, lo_flat, gidx_flat)


def _tile_lanes(x, width):
    t = x
    while t.shape[1] < width:
        t = jnp.concatenate([t, t], axis=1)
    return t


def _qkv_body(lens_ref, hs_ref, pos_ref, w_ref, g_ref, q_ref, k_ref, v_ref):
    b = pl.program_id(0)
    qi = pl.program_id(1)
    ln = lens_ref[b, 0]

    @pl.when(qi * _BQ < ln)
    def _():
        x = hs_ref[0]
        var = jnp.mean(x * x, axis=-1, keepdims=True)
        xn = (x * lax.rsqrt(var + _EPS)) * g_ref[0]
        qkv = jnp.dot(xn.astype(jnp.bfloat16), w_ref[...],
                      preferred_element_type=jnp.float32)
        pos = pos_ref[0].astype(jnp.float32) - b * float(_S)
        j32 = lax.broadcasted_iota(jnp.int32, (1, 32), 1).astype(jnp.float32)
        invf = jnp.exp(j32 * (-np.log(_THETA) / 32.0))
        ang = pos * invf
        c = _tile_lanes(jnp.cos(ang), _H)
        s = _tile_lanes(jnp.sin(ang), _H)
        l_idx = lax.broadcasted_iota(jnp.int32, (1, _H), 1)
        sel = (l_idx % 64) < 32

        def rope(t):
            xp = jnp.concatenate([t[:, 32:], t[:, :32]], axis=1)
            xm = jnp.concatenate([t[:, -32:], t[:, :-32]], axis=1)
            return jnp.where(sel, -xp, xm)

        qp = qkv[:, :_H]
        kp = qkv[:, _H:2 * _H]
        q_ref[0] = (qp * c + rope(qp) * s).astype(jnp.bfloat16)
        k_ref[0] = (kp * c + rope(kp) * s).astype(jnp.bfloat16)
        v_ref[0] = qkv[:, 2 * _H:].astype(jnp.bfloat16)

    @pl.when(qi * _BQ >= ln)
    def _():
        z = jnp.zeros((_BQ, _H), jnp.bfloat16)
        q_ref[0] = z
        k_ref[0] = z
        v_ref[0] = z


def _qkv_call(lens_x, hs_c, pos3, wqkv, g1):
    grid_spec = pltpu.PrefetchScalarGridSpec(
        num_scalar_prefetch=1,
        grid=(_B, _NQ),
        in_specs=[
            pl.BlockSpec((1, _BQ, _H), lambda b, qi, L: (b, qi, 0)),
            pl.BlockSpec((1, _BQ, 1), lambda b, qi, L: (b * _NQ + qi, 0, 0)),
            pl.BlockSpec((_H, 3 * _H), lambda b, qi, L: (0, 0)),
            pl.BlockSpec((1, _H), lambda b, qi, L: (0, 0)),
        ],
        out_specs=[
            pl.BlockSpec((1, _BQ, _H), lambda b, qi, L: (b, qi, 0)),
            pl.BlockSpec((1, _BQ, _H), lambda b, qi, L: (b, qi, 0)),
            pl.BlockSpec((1, _BQ, _H), lambda b, qi, L: (b, qi, 0)),
        ],
    )
    shp = jax.ShapeDtypeStruct((_B, _S, _H), jnp.bfloat16)
    return pl.pallas_call(
        _qkv_body,
        grid_spec=grid_spec,
        out_shape=[shp, shp, shp],
        compiler_params=pltpu.CompilerParams(
            dimension_semantics=("parallel", "parallel")),
        interpret=_INTERPRET,
    )(lens_x, hs_c, pos3, wqkv, g1)


def _attn_body(lens_ref, q_ref, k_ref, v_ref, o_ref, k0s, k1s, v0s, v1s):
    b = pl.program_id(0)
    ln = lens_ref[b, 0]
    scale = 1.0 / np.sqrt(_HD)

    k0s[...] = k_ref[0][:, :_HD]
    k1s[...] = k_ref[0][:, _HD:]
    v0s[...] = v_ref[0][:, :_HD]
    v1s[...] = v_ref[0][:, _HD:]

    def upd(s, m, l, acc, vblk):
        m_new = jnp.maximum(m, jnp.max(s, axis=1, keepdims=True))
        alpha = jnp.exp(m - m_new)
        p = jnp.exp(s - m_new)
        l_new = l * alpha + jnp.sum(p, axis=1, keepdims=True)
        acc_new = acc * alpha + jnp.dot(p.astype(jnp.bfloat16), vblk,
                                        preferred_element_type=jnp.float32)
        return m_new, l_new, acc_new

    for qi in range(_NQ):
        start = qi * _BQ

        @pl.when(start < ln)
        def _(qi=qi, start=start):
            qq = q_ref[0, pl.ds(start, _BQ), :]
            q0 = qq[:, :_HD]
            q1 = qq[:, _HD:]

            def blockstep(kb, carry, masked):
                m0, l0, a0, m1, l1, a1 = carry
                kb0 = k0s[pl.ds(kb * _BK, _BK), :]
                kb1 = k1s[pl.ds(kb * _BK, _BK), :]
                vb0 = v0s[pl.ds(kb * _BK, _BK), :]
                vb1 = v1s[pl.ds(kb * _BK, _BK), :]
                s0 = lax.dot_general(q0, kb0, (((1,), (1,)), ((), ())),
                                     preferred_element_type=jnp.float32)
                s1 = lax.dot_general(q1, kb1, (((1,), (1,)), ((), ())),
                                     preferred_element_type=jnp.float32)
                s0 = s0 * scale
                s1 = s1 * scale
                if masked:
                    row = start + lax.broadcasted_iota(jnp.int32, (_BQ, 1), 0)
                    col = kb * _BK + lax.broadcasted_iota(
                        jnp.int32, (1, _BK), 1)
                    ok = col <= row
                    s0 = jnp.where(ok, s0, -1e30)
                    s1 = jnp.where(ok, s1, -1e30)
                m0, l0, a0 = upd(s0, m0, l0, a0, vb0)
                m1, l1, a1 = upd(s1, m1, l1, a1, vb1)
                return m0, l0, a0, m1, l1, a1

            mi = jnp.full((_BQ, 1), -1e30, jnp.float32)
            li = jnp.zeros((_BQ, 1), jnp.float32)
            ai = jnp.zeros((_BQ, _HD), jnp.float32)
            carry = (mi, li, ai, mi, li, ai)
            for kb in range(qi):
                carry = blockstep(kb, carry, False)
            m0, l0, a0, m1, l1, a1 = blockstep(qi, carry, True)
            o_ref[0, pl.ds(start, _BQ), :] = jnp.concatenate(
                [(a0 / l0), (a1 / l1)], axis=1).astype(jnp.bfloat16)


def _attn_call(lens_x, q, k, v):
    grid_spec = pltpu.PrefetchScalarGridSpec(
        num_scalar_prefetch=1,
        grid=(_B, _NH // 2),
        in_specs=[
            pl.BlockSpec((1, _S, 2 * _HD), lambda b, h, L: (b, 0, h)),
            pl.BlockSpec((1, _S, 2 * _HD), lambda b, h, L: (b, 0, h)),
            pl.BlockSpec((1, _S, 2 * _HD), lambda b, h, L: (b, 0, h)),
        ],
        out_specs=pl.BlockSpec((1, _S, 2 * _HD),
                               lambda b, h, L: (b, 0, h)),
        scratch_shapes=[
            pltpu.VMEM((_S, _HD), jnp.bfloat16),
            pltpu.VMEM((_S, _HD), jnp.bfloat16),
            pltpu.VMEM((_S, _HD), jnp.bfloat16),
            pltpu.VMEM((_S, _HD), jnp.bfloat16),
        ],
    )
    return pl.pallas_call(
        _attn_body,
        grid_spec=grid_spec,
        out_shape=jax.ShapeDtypeStruct((_B, _S, _H), jnp.bfloat16),
        compiler_params=pltpu.CompilerParams(
            dimension_semantics=("parallel", "parallel")),
        interpret=_INTERPRET,
    )(lens_x, q, k, v)


def _mlp_body(lens_ref, a_ref, hs_ref, wo_ref, g2_ref, wg_ref, wu_ref, wd_ref,
              o_ref):
    b = pl.program_id(0)
    qi = pl.program_id(1)
    ln = lens_ref[b, 0]

    @pl.when(qi * _BQ < ln)
    def _():
        r2 = hs_ref[0] + jnp.dot(a_ref[0], wo_ref[...],
                                 preferred_element_type=jnp.float32)
        var = jnp.mean(r2 * r2, axis=-1, keepdims=True)
        xn = ((r2 * lax.rsqrt(var + _EPS)) * g2_ref[0]).astype(jnp.bfloat16)
        g = jnp.dot(xn, wg_ref[...], preferred_element_type=jnp.float32)
        u = jnp.dot(xn, wu_ref[...], preferred_element_type=jnp.float32)
        act = (g * jax.nn.sigmoid(g) * u).astype(jnp.bfloat16)
        o_ref[0] = r2 + jnp.dot(act, wd_ref[...],
                                preferred_element_type=jnp.float32)


def _mlp_call(lens_x, attn, hs_c, wo, g2, wg, wu, wd):
    grid_spec = pltpu.PrefetchScalarGridSpec(
        num_scalar_prefetch=1,
        grid=(_B, _NQ),
        in_specs=[
            pl.BlockSpec((1, _BQ, _H), lambda b, qi, L: (b, qi, 0)),
            pl.BlockSpec((1, _BQ, _H), lambda b, qi, L: (b, qi, 0)),
            pl.BlockSpec((_H, _H), lambda b, qi, L: (0, 0)),
            pl.BlockSpec((1, _H), lambda b, qi, L: (0, 0)),
            pl.BlockSpec((_H, _F), lambda b, qi, L: (0, 0)),
            pl.BlockSpec((_H, _F), lambda b, qi, L: (0, 0)),
            pl.BlockSpec((_F, _H), lambda b, qi, L: (0, 0)),
        ],
        out_specs=pl.BlockSpec((1, _BQ, _H), lambda b, qi, L: (b, qi, 0)),
    )
    return pl.pallas_call(
        _mlp_body,
        grid_spec=grid_spec,
        out_shape=jax.ShapeDtypeStruct((_B, _S, _H), jnp.float32),
        compiler_params=pltpu.CompilerParams(
            dimension_semantics=("parallel", "parallel")),
        interpret=_INTERPRET,
    )(lens_x, attn, hs_c, wo, g2, wg, wu, wd)


def kernel(hidden_states, position_ids, topk_mask, topk_scores, g1, g2,
           Wq, Wk, Wv, Wo, Wg, Wu, Wd):
    mask_i = topk_mask.astype(jnp.int32)
    gidx, lens_x = _sc_index_build(mask_i)

    hid_flat = hidden_states.reshape(_B * _S, _H)
    hs_c_flat = _sc_gather(hid_flat, gidx.reshape(-1))
    hs_c = hs_c_flat.reshape(_B, _S, _H)

    pos3 = gidx.reshape(_B * _NQ, _BQ, 1)
    wqkv = jnp.concatenate([Wq, Wk, Wv], axis=1).astype(jnp.bfloat16)
    q, k, v = _qkv_call(lens_x, hs_c, pos3, wqkv, g1.reshape(1, _H))

    attn = _attn_call(lens_x, q, k, v)

    layer_out = _mlp_call(lens_x, attn, hs_c,
                          Wo.astype(jnp.bfloat16), g2.reshape(1, _H),
                          Wg.astype(jnp.bfloat16), Wu.astype(jnp.bfloat16),
                          Wd.astype(jnp.bfloat16))

    outp = _sc_scatter(hid_flat, layer_out.reshape(_B * _S, _H),
                       gidx.reshape(-1))
    return outp[:_B * _S].reshape(_B, _S, _H)


# speedup vs baseline: 2.6512x; 1.0175x over previous
import functools

import numpy as np

import jax
import jax.numpy as jnp
from jax import lax
from jax.experimental import pallas as pl
from jax.experimental.pallas import tpu as pltpu
from jax.experimental.pallas import tpu_sc as plsc

_B, _S, _H, _NH, _HD, _F = 2, 4096, 1024, 16, 64, 2816
_EPS = 1e-5
_THETA = 10000.0
_BQ = 512
_BK = 512
_NQ = _S // _BQ
_TRASH = _B * _S
_NTILES = 32
_RPT = _B * _S // _NTILES
_SUB = 32
_NCH = _RPT // _SUB

_INTERPRET = False


def _sc_index_build(mask_i32):
    mesh = plsc.VectorSubcoreMesh(core_axis_name="c", subcore_axis_name="s", num_cores=2, num_subcores=16)

    @functools.partial(
        pl.kernel,
        out_type=(
            jax.ShapeDtypeStruct((_B, _S), jnp.int32),
            jax.ShapeDtypeStruct((_B, 16), jnp.int32),
        ),
        mesh=mesh,
        scratch_types=[
            pltpu.VMEM((_S,), jnp.int32),
            pltpu.VMEM((_S,), jnp.int32),
            pltpu.VMEM((16,), jnp.int32),
        ],
        compiler_params=pltpu.CompilerParams(needs_layout_passes=False),
        interpret=_INTERPRET,
    )
    def k(mask_hbm, gidx_hbm, lens_hbm, mask_v, gidx_v, lens_v):
        wid = lax.axis_index("s") * 2 + lax.axis_index("c")

        @pl.when(wid == 0)
        def _():
            def batch_body(b, _):
                pltpu.sync_copy(mask_hbm.at[b], mask_v)
                neg1 = jnp.full((16,), -1, jnp.int32)

                def initb(i, c):
                    gidx_v[pl.ds(i * 16, 16)] = neg1
                    return c

                lax.fori_loop(0, _S // 16, initb, 0)
                base = b * _S

                def chunk(i, carry):
                    m = mask_v[pl.ds(i * 16, 16)]
                    mb = m != 0
                    c = plsc.cumsum(m)
                    rank = c - 1 + carry
                    tvec = lax.iota(jnp.int32, 16) + i * 16 + base
                    plsc.store_scatter(gidx_v, [rank], tvec, mask=mb)
                    return carry + jnp.sum(m)

                ln = lax.fori_loop(0, _S // 16, chunk, jnp.int32(0))
                pltpu.sync_copy(gidx_v, gidx_hbm.at[b])
                lens_v[...] = jnp.zeros((16,), jnp.int32) + ln
                pltpu.sync_copy(lens_v, lens_hbm.at[b])
                return 0

            lax.fori_loop(0, _B, batch_body, 0)

    return k(mask_i32)


def _sc_gather(hid_flat, gidx_flat):
    mesh = plsc.VectorSubcoreMesh(core_axis_name="c", subcore_axis_name="s", num_cores=2, num_subcores=16)

    @functools.partial(
        pl.kernel,
        out_type=jax.ShapeDtypeStruct((_B * _S, _H), jnp.float32),
        mesh=mesh,
        scratch_types=[
            pltpu.VMEM((_RPT,), jnp.int32),
            pltpu.VMEM((_SUB, _H), jnp.float32),
            pltpu.VMEM((_SUB, _H), jnp.float32),
            pltpu.VMEM((_SUB, _H), jnp.float32),
            pltpu.SemaphoreType.DMA,
            pltpu.SemaphoreType.DMA,
            pltpu.SemaphoreType.DMA,
            pltpu.SemaphoreType.DMA,
            pltpu.SemaphoreType.DMA,
            pltpu.SemaphoreType.DMA,
        ],
        compiler_params=pltpu.CompilerParams(needs_layout_passes=False),
        interpret=_INTERPRET,
    )
    def k(hid_hbm, gidx_hbm, out_hbm, idx_all, buf0, buf1, buf2,
          sg0, sg1, sg2, sw0, sw1, sw2):
        wid = lax.axis_index("s") * 2 + lax.axis_index("c")
        base = wid * _RPT
        bbase = (base // _S) * _S
        pltpu.sync_copy(gidx_hbm.at[pl.ds(base, _RPT)], idx_all)
        n = jnp.int32(0)
        for t in range(_RPT // 16):
            g = idx_all[pl.ds(t * 16, 16)]
            n = n + jnp.sum((g >= 0).astype(jnp.int32))
            idx_all[pl.ds(t * 16, 16)] = jnp.where(g < 0, bbase, g)
        bufs = (buf0, buf1, buf2)
        sgs = (sg0, sg1, sg2)
        sws = (sw0, sw1, sw2)

        def g_desc(j):
            return pltpu.make_async_copy(
                hid_hbm.at[idx_all.at[pl.ds(j * _SUB, _SUB)]],
                bufs[j % 3], sgs[j % 3])

        def w_desc(j):
            return pltpu.make_async_copy(
                bufs[j % 3], out_hbm.at[pl.ds(base + j * _SUB, _SUB)],
                sws[j % 3])

        for j in range(3):
            @pl.when(j * _SUB < n)
            def _(j=j):
                g_desc(j).start()
        for j in range(_NCH):
            @pl.when(j * _SUB < n)
            def _(j=j):
                g_desc(j).wait()
                w_desc(j).start()
            if j + 3 < _NCH:
                @pl.when((j + 3) * _SUB < n)
                def _(j=j):
                    w_desc(j).wait()
                    g_desc(j + 3).start()
        for j in range(_NCH):
            if j + 3 < _NCH:
                tail = (j * _SUB < n) & ((j + 3) * _SUB >= n)
            else:
                tail = j * _SUB < n

            @pl.when(tail)
            def _(j=j):
                w_desc(j).wait()

    return k(hid_flat, gidx_flat)


def _sc_scatter(hid_flat, lo_flat, gidx_flat):
    mesh = plsc.VectorSubcoreMesh(core_axis_name="c", subcore_axis_name="s", num_cores=2, num_subcores=16)

    @functools.partial(
        pl.kernel,
        out_type=jax.ShapeDtypeStruct((_B * _S + 8, _H), jnp.float32),
        mesh=mesh,
        scratch_types=[
            pltpu.VMEM((_S + _SUB,), jnp.int32),
            pltpu.VMEM((_SUB,), jnp.int32),
            pltpu.VMEM((_SUB,), jnp.int32),
            pltpu.VMEM((_SUB,), jnp.int32),
            pltpu.VMEM((_SUB, _H), jnp.float32),
            pltpu.VMEM((_SUB, _H), jnp.float32),
            pltpu.VMEM((_SUB, _H), jnp.float32),
            pltpu.SemaphoreType.DMA,
            pltpu.SemaphoreType.DMA,
            pltpu.SemaphoreType.DMA,
            pltpu.SemaphoreType.DMA,
            pltpu.SemaphoreType.DMA,
            pltpu.SemaphoreType.DMA,
        ],
        compiler_params=pltpu.CompilerParams(needs_layout_passes=False),
        interpret=_INTERPRET,
    )
    def k(hid_hbm, lo_hbm, gidx_hbm, out_hbm, gv, ib0, ib1, ib2,
          buf0, buf1, buf2, sl0, sl1, sl2, ss0, ss1, ss2):
        wid = lax.axis_index("s") * 2 + lax.axis_index("c")
        base = wid * _RPT
        bidx = base // _S
        bbase = bidx * _S
        bufs = (buf0, buf1, buf2)
        ibs = (ib0, ib1, ib2)
        sls = (sl0, sl1, sl2)
        sss = (ss0, ss1, ss2)

        def bl_desc(j):
            return pltpu.make_async_copy(
                hid_hbm.at[pl.ds(base + j * _SUB, _SUB)],
                bufs[j % 3], sls[j % 3])

        def bw_desc(j):
            return pltpu.make_async_copy(
                bufs[j % 3], out_hbm.at[pl.ds(base + j * _SUB, _SUB)],
                sss[j % 3])

        for j in range(3):
            bl_desc(j).start()
        for j in range(_NCH):
            bl_desc(j).wait()
            bw_desc(j).start()
            if j + 3 < _NCH:
                bw_desc(j).wait()
                bl_desc(j + 3).start()
        for j in range(_NCH - 3, _NCH):
            bw_desc(j).wait()

        pltpu.sync_copy(gidx_hbm.at[pl.ds(bbase, _S)], gv.at[pl.ds(0, _S)])

        def cnt(i, carry):
            lo, hi = carry
            g = gv[pl.ds(i * 16, 16)]
            ok = g >= 0
            lo = lo + jnp.sum((ok & (g < base)).astype(jnp.int32))
            hi = hi + jnp.sum((ok & (g < base + _RPT)).astype(jnp.int32))
            return lo, hi

        r_lo, r_hi = lax.fori_loop(0, _S // 16, cnt,
                                   (jnp.int32(0), jnp.int32(0)))
        r8 = (r_lo // 8) * 8

        _NJ = _NCH + 1

        def rs_of(j):
            return jnp.minimum(r8 + j * _SUB, _S - _SUB)

        def l_desc(j):
            return pltpu.make_async_copy(
                lo_hbm.at[pl.ds(bbase + rs_of(j), _SUB)],
                bufs[j % 3], sls[j % 3])

        def s_desc(j):
            return pltpu.make_async_copy(
                bufs[j % 3], out_hbm.at[ibs[j % 3]], sss[j % 3])

        def build_idx(j):
            rs = rs_of(j)
            for t in range(_SUB // 16):
                g = gv[pl.ds(rs + t * 16, 16)]
                lane_r = lax.iota(jnp.int32, 16) + (rs + t * 16)
                ibs[j % 3][pl.ds(t * 16, 16)] = jnp.where(
                    (g < 0) | (lane_r >= r_hi), _TRASH, g)

        def act(j):
            return r8 + j * _SUB < r_hi

        for j in range(3):
            @pl.when(act(j))
            def _(j=j):
                build_idx(j)
                l_desc(j).start()
        for j in range(_NJ):
            @pl.when(act(j))
            def _(j=j):
                l_desc(j).wait()
                s_desc(j).start()
            if j + 3 < _NJ:
                @pl.when(act(j + 3))
                def _(j=j):
                    s_desc(j).wait()
                    build_idx(j + 3)
                    l_desc(j + 3).start()
        for j in range(_NJ):
            if j + 3 < _NJ:
                tail = act(j) & jnp.logical_not(act(j + 3))
            else:
                tail = act(j)

            @pl.when(tail)
            def _(j=j):
                s_desc(j).wait()

    return k(hid_flat, lo_flat, gidx_flat)


def _tile_lanes(x, width):
    t = x
    while t.shape[1] < width:
        t = jnp.concatenate([t, t], axis=1)
    return t


def _qkv_body(lens_ref, hs_ref, pos_ref, w_ref, g_ref, q_ref, k_ref, v_ref):
    b = pl.program_id(0)
    qi = pl.program_id(1)
    ln = lens_ref[b, 0]

    @pl.when(qi * _BQ < ln)
    def _():
        x = hs_ref[0]
        var = jnp.mean(x * x, axis=-1, keepdims=True)
        xn = (x * lax.rsqrt(var + _EPS)) * g_ref[0]
        qkv = jnp.dot(xn.astype(jnp.bfloat16), w_ref[...],
                      preferred_element_type=jnp.float32)
        pos = pos_ref[0].astype(jnp.float32) - b * float(_S)
        j32 = lax.broadcasted_iota(jnp.int32, (1, 32), 1).astype(jnp.float32)
        invf = jnp.exp(j32 * (-np.log(_THETA) / 32.0))
        ang = pos * invf
        c = _tile_lanes(jnp.cos(ang), _H)
        s = _tile_lanes(jnp.sin(ang), _H)
        l_idx = lax.broadcasted_iota(jnp.int32, (1, _H), 1)
        sel = (l_idx % 64) < 32

        def rope(t):
            xp = jnp.concatenate([t[:, 32:], t[:, :32]], axis=1)
            xm = jnp.concatenate([t[:, -32:], t[:, :-32]], axis=1)
            return jnp.where(sel, -xp, xm)

        qp = qkv[:, :_H]
        kp = qkv[:, _H:2 * _H]
        q_ref[0] = (qp * c + rope(qp) * s).astype(jnp.bfloat16)
        k_ref[0] = (kp * c + rope(kp) * s).astype(jnp.bfloat16)
        v_ref[0] = qkv[:, 2 * _H:].astype(jnp.bfloat16)


def _qkv_call(lens_x, hs_c, pos3, wqkv, g1):
    grid_spec = pltpu.PrefetchScalarGridSpec(
        num_scalar_prefetch=1,
        grid=(_B, _NQ),
        in_specs=[
            pl.BlockSpec((1, _BQ, _H), lambda b, qi, L: (b, qi, 0)),
            pl.BlockSpec((1, _BQ, 1), lambda b, qi, L: (b * _NQ + qi, 0, 0)),
            pl.BlockSpec((_H, 3 * _H), lambda b, qi, L: (0, 0)),
            pl.BlockSpec((1, _H), lambda b, qi, L: (0, 0)),
        ],
        out_specs=[
            pl.BlockSpec((1, _BQ, _H), lambda b, qi, L: (b, qi, 0)),
            pl.BlockSpec((1, _BQ, _H), lambda b, qi, L: (b, qi, 0)),
            pl.BlockSpec((1, _BQ, _H), lambda b, qi, L: (b, qi, 0)),
        ],
    )
    shp = jax.ShapeDtypeStruct((_B, _S, _H), jnp.bfloat16)
    return pl.pallas_call(
        _qkv_body,
        grid_spec=grid_spec,
        out_shape=[shp, shp, shp],
        compiler_params=pltpu.CompilerParams(
            dimension_semantics=("parallel", "parallel")),
        interpret=_INTERPRET,
    )(lens_x, hs_c, pos3, wqkv, g1)


def _attn_body(lens_ref, q_ref, k_ref, v_ref, o_ref, k0s, k1s, v0s, v1s):
    b = pl.program_id(0)
    ln = lens_ref[b, 0]
    scale = 1.0 / np.sqrt(_HD)

    k0s[...] = k_ref[0][:, :_HD]
    k1s[...] = k_ref[0][:, _HD:]
    v0s[...] = v_ref[0][:, :_HD]
    v1s[...] = v_ref[0][:, _HD:]

    def upd(s, m, l, acc, vblk):
        m_new = jnp.maximum(m, jnp.max(s, axis=1, keepdims=True))
        alpha = jnp.exp(m - m_new)
        p = jnp.exp(s - m_new)
        l_new = l * alpha + jnp.sum(p, axis=1, keepdims=True)
        acc_new = acc * alpha + jnp.dot(p.astype(jnp.bfloat16), vblk,
                                        preferred_element_type=jnp.float32)
        return m_new, l_new, acc_new

    for qi in range(_NQ):
        start = qi * _BQ

        @pl.when(start < ln)
        def _(qi=qi, start=start):
            qq = q_ref[0, pl.ds(start, _BQ), :]
            q0 = qq[:, :_HD] * jnp.bfloat16(scale)
            q1 = qq[:, _HD:] * jnp.bfloat16(scale)

            def blockstep(kb, carry, masked):
                m0, l0, a0, m1, l1, a1 = carry
                kb0 = k0s[pl.ds(kb * _BK, _BK), :]
                kb1 = k1s[pl.ds(kb * _BK, _BK), :]
                vb0 = v0s[pl.ds(kb * _BK, _BK), :]
                vb1 = v1s[pl.ds(kb * _BK, _BK), :]
                s0 = lax.dot_general(q0, kb0, (((1,), (1,)), ((), ())),
                                     preferred_element_type=jnp.float32)
                s1 = lax.dot_general(q1, kb1, (((1,), (1,)), ((), ())),
                                     preferred_element_type=jnp.float32)
                if masked:
                    row = start + lax.broadcasted_iota(jnp.int32, (_BQ, 1), 0)
                    col = kb * _BK + lax.broadcasted_iota(
                        jnp.int32, (1, _BK), 1)
                    ok = col <= row
                    s0 = jnp.where(ok, s0, -1e30)
                    s1 = jnp.where(ok, s1, -1e30)
                m0, l0, a0 = upd(s0, m0, l0, a0, vb0)
                m1, l1, a1 = upd(s1, m1, l1, a1, vb1)
                return m0, l0, a0, m1, l1, a1

            mi = jnp.full((_BQ, 1), -1e30, jnp.float32)
            li = jnp.zeros((_BQ, 1), jnp.float32)
            ai = jnp.zeros((_BQ, _HD), jnp.float32)
            carry = (mi, li, ai, mi, li, ai)
            for kb in range(qi):
                carry = blockstep(kb, carry, False)
            m0, l0, a0, m1, l1, a1 = blockstep(qi, carry, True)
            o_ref[0, pl.ds(start, _BQ), :] = jnp.concatenate(
                [(a0 / l0), (a1 / l1)], axis=1).astype(jnp.bfloat16)


def _attn_call(lens_x, q, k, v):
    grid_spec = pltpu.PrefetchScalarGridSpec(
        num_scalar_prefetch=1,
        grid=(_B, _NH // 2),
        in_specs=[
            pl.BlockSpec((1, _S, 2 * _HD), lambda b, h, L: (b, 0, h)),
            pl.BlockSpec((1, _S, 2 * _HD), lambda b, h, L: (b, 0, h)),
            pl.BlockSpec((1, _S, 2 * _HD), lambda b, h, L: (b, 0, h)),
        ],
        out_specs=pl.BlockSpec((1, _S, 2 * _HD),
                               lambda b, h, L: (b, 0, h)),
        scratch_shapes=[
            pltpu.VMEM((_S, _HD), jnp.bfloat16),
            pltpu.VMEM((_S, _HD), jnp.bfloat16),
            pltpu.VMEM((_S, _HD), jnp.bfloat16),
            pltpu.VMEM((_S, _HD), jnp.bfloat16),
        ],
    )
    return pl.pallas_call(
        _attn_body,
        grid_spec=grid_spec,
        out_shape=jax.ShapeDtypeStruct((_B, _S, _H), jnp.bfloat16),
        compiler_params=pltpu.CompilerParams(
            dimension_semantics=("parallel", "parallel")),
        interpret=_INTERPRET,
    )(lens_x, q, k, v)


def _mlp_body(lens_ref, a_ref, hs_ref, wo_ref, g2_ref, wg_ref, wu_ref, wd_ref,
              o_ref):
    b = pl.program_id(0)
    qi = pl.program_id(1)
    ln = lens_ref[b, 0]

    @pl.when(qi * _BQ < ln)
    def _():
        r2 = hs_ref[0] + jnp.dot(a_ref[0], wo_ref[...],
                                 preferred_element_type=jnp.float32)
        var = jnp.mean(r2 * r2, axis=-1, keepdims=True)
        xn = ((r2 * lax.rsqrt(var + _EPS)) * g2_ref[0]).astype(jnp.bfloat16)
        g = jnp.dot(xn, wg_ref[...], preferred_element_type=jnp.float32)
        u = jnp.dot(xn, wu_ref[...], preferred_element_type=jnp.float32)
        act = (g * jax.nn.sigmoid(g) * u).astype(jnp.bfloat16)
        o_ref[0] = r2 + jnp.dot(act, wd_ref[...],
                                preferred_element_type=jnp.float32)


def _mlp_call(lens_x, attn, hs_c, wo, g2, wg, wu, wd):
    grid_spec = pltpu.PrefetchScalarGridSpec(
        num_scalar_prefetch=1,
        grid=(_B, _NQ),
        in_specs=[
            pl.BlockSpec((1, _BQ, _H), lambda b, qi, L: (b, qi, 0)),
            pl.BlockSpec((1, _BQ, _H), lambda b, qi, L: (b, qi, 0)),
            pl.BlockSpec((_H, _H), lambda b, qi, L: (0, 0)),
            pl.BlockSpec((1, _H), lambda b, qi, L: (0, 0)),
            pl.BlockSpec((_H, _F), lambda b, qi, L: (0, 0)),
            pl.BlockSpec((_H, _F), lambda b, qi, L: (0, 0)),
            pl.BlockSpec((_F, _H), lambda b, qi, L: (0, 0)),
        ],
        out_specs=pl.BlockSpec((1, _BQ, _H), lambda b, qi, L: (b, qi, 0)),
    )
    return pl.pallas_call(
        _mlp_body,
        grid_spec=grid_spec,
        out_shape=jax.ShapeDtypeStruct((_B, _S, _H), jnp.float32),
        compiler_params=pltpu.CompilerParams(
            dimension_semantics=("parallel", "parallel")),
        interpret=_INTERPRET,
    )(lens_x, attn, hs_c, wo, g2, wg, wu, wd)


def kernel(hidden_states, position_ids, topk_mask, topk_scores, g1, g2,
           Wq, Wk, Wv, Wo, Wg, Wu, Wd):
    mask_i = topk_mask.astype(jnp.int32)
    gidx, lens_x = _sc_index_build(mask_i)

    hid_flat = hidden_states.reshape(_B * _S, _H)
    hs_c_flat = _sc_gather(hid_flat, gidx.reshape(-1))
    hs_c = hs_c_flat.reshape(_B, _S, _H)

    pos3 = gidx.reshape(_B * _NQ, _BQ, 1)
    wqkv = jnp.concatenate([Wq, Wk, Wv], axis=1).astype(jnp.bfloat16)
    q, k, v = _qkv_call(lens_x, hs_c, pos3, wqkv, g1.reshape(1, _H))

    attn = _attn_call(lens_x, q, k, v)

    layer_out = _mlp_call(lens_x, attn, hs_c,
                          Wo.astype(jnp.bfloat16), g2.reshape(1, _H),
                          Wg.astype(jnp.bfloat16), Wu.astype(jnp.bfloat16),
                          Wd.astype(jnp.bfloat16))

    outp = _sc_scatter(hid_flat, layer_out.reshape(_B * _S, _H),
                       gidx.reshape(-1))
    return outp[:_B * _S].reshape(_B, _S, _H)
